# baseline (speedup 1.0000x reference)
import functools

import jax
import jax.numpy as jnp
from jax import lax
from jax.experimental import pallas as pl
from jax.experimental.pallas import tpu as pltpu, tpu_sc as plsc

EPS = 1e-05
H = 128
CH = 80

_SC_PARAMS = pltpu.CompilerParams(needs_layout_passes=False)


def _mesh():
    return plsc.VectorSubcoreMesh(core_axis_name="c", subcore_axis_name="s")


def _f32(shape):
    return jax.ShapeDtypeStruct(shape, jnp.float32)


def _sc_gather2(table, i0_3d, i1_3d, E):
    CPW = E // (32 * CH)

    @functools.partial(
        pl.kernel,
        out_type=(_f32((E, H)), _f32((E, H))),
        mesh=_mesh(),
        compiler_params=_SC_PARAMS,
        scratch_types=[
            pltpu.VMEM((CPW, CH), jnp.int32),
            pltpu.VMEM((CPW, CH), jnp.int32),
            pltpu.VMEM((CH, H), jnp.float32),
            pltpu.VMEM((CH, H), jnp.float32),
            pltpu.SemaphoreType.DMA,
            pltpu.SemaphoreType.DMA,
        ],
    )
    def k(tab_h, i0_h, i1_h, g0_h, g1_h, i0v, i1v, b0, b1, s0, s1):
        cid = lax.axis_index("c")
        sid = lax.axis_index("s")
        w = sid * 2 + cid
        r0 = w * CPW
        pltpu.sync_copy(i0_h.at[w], i0v)
        pltpu.sync_copy(i1_h.at[w], i1v)

        def body(ch, _):
            cp0 = pltpu.async_copy(tab_h.at[i0v.at[ch]], b0, s0)
            cp1 = pltpu.async_copy(tab_h.at[i1v.at[ch]], b1, s1)
            cp0.wait()
            cp1.wait()
            base = (r0 + ch) * CH
            pltpu.sync_copy(b0, g0_h.at[pl.ds(base, CH)])
            pltpu.sync_copy(b1, g1_h.at[pl.ds(base, CH)])
            return 0

        lax.fori_loop(0, CPW, body, 0)

    return k(table, i0_3d, i1_3d)


def _sc_scatter_nodes(h1, i0_3d, i1_3d, E, N):
    CPW = E // (32 * CH)
    NP = 10240
    NH = N // 2
    ACC = 5120
    STR = ACC // 16
    ZB = STR // 4

    @functools.partial(
        pl.kernel,
        out_type=_f32((2, NP, H)),
        mesh=_mesh(),
        compiler_params=_SC_PARAMS,
        scratch_types=[
            pltpu.VMEM((CPW, CH), jnp.int32),
            pltpu.VMEM((CPW, CH), jnp.int32),
            pltpu.VMEM((2, CH), jnp.int32),
            pltpu.VMEM((CH, H), jnp.float32),
            pltpu.VMEM((ZB, H), jnp.float32),
            pltpu.VMEM_SHARED((ACC + 8, H), jnp.float32),
            pltpu.SemaphoreType.DMA,
        ],
    )
    def k(h1_h, i0_h, i1_h, out_h, i0v, i1v, idxb, hbuf, zbuf, nacc, sem):
        cid = lax.axis_index("c")
        sid = lax.axis_index("s")
        w = sid * 2 + cid
        r0 = w * CPW
        zv = jnp.zeros((16,), jnp.float32)

        def zrow(r, _):
            for j in range(H // 16):
                zbuf[r, pl.ds(16 * j, 16)] = zv
            return 0

        lax.fori_loop(0, ZB, zrow, 0)
        pltpu.sync_copy(i0_h.at[w], i0v)
        pltpu.sync_copy(i1_h.at[w], i1v)

        for half in (0, 1):
            if half:
                plsc.subcore_barrier()
            for b in range(4):
                pltpu.sync_copy(zbuf, nacc.at[pl.ds(sid * STR + b * ZB, ZB)])

            @pl.when(sid == 0)
            def _():
                pltpu.sync_copy(zbuf.at[pl.ds(0, 8)], nacc.at[pl.ds(ACC, 8)])

            plsc.subcore_barrier()
            nbase = half * NH

            def body(ch, _):
                base = (r0 + ch) * CH
                pltpu.sync_copy(h1_h.at[pl.ds(base, CH)], hbuf)
                for i5 in range(CH // 16):
                    sl = pl.ds(16 * i5, 16)
                    v0 = i0v[ch, sl] - nbase
                    v1 = i1v[ch, sl] - nbase
                    idxb[0, sl] = jnp.where((v0 >= 0) & (v0 < NH), v0, ACC)
                    idxb[1, sl] = jnp.where((v1 >= 0) & (v1 < NH), v1, ACC)
                pltpu.sync_copy(hbuf, nacc.at[idxb.at[0]], add=True)
                pltpu.sync_copy(hbuf, nacc.at[idxb.at[1]], add=True)
                return 0

            lax.fori_loop(0, CPW, body, 0)
            plsc.subcore_barrier()
            for b in range(4):
                rr = sid * STR + b * ZB
                pltpu.sync_copy(nacc.at[pl.ds(rr, ZB)],
                                out_h.at[cid, pl.ds(nbase + rr, ZB)])

    return k(h1, i0_3d, i1_3d)


def _sc_seg_means(dom3d, C, T2, gather_src=None, cee3d=None, linear_src=None,
                  rcp_in=None):
    NCHK = T2 // CH
    NPT = NCHK // 16
    SEG = C // 2
    SEGH = SEG // 2
    SEGP = 10240
    ACC = 5120
    STR = ACC // 16
    ZB = STR // 4
    do_g = gather_src is not None
    do_l = linear_src is not None
    do_cnt = rcp_in is None

    outs = []
    if do_g:
        outs.append(_f32((T2, H)))
        outs.append(_f32((2 * SEGP, H)))
    if do_l:
        outs.append(_f32((2 * SEGP, H)))
    if do_cnt:
        outs.append(_f32((2 * SEGP, 16)))

    ins = [dom3d]
    if do_g:
        ins += [gather_src, cee3d]
    if do_l:
        ins += [linear_src]
    if not do_cnt:
        ins += [rcp_in]

    @functools.partial(
        pl.kernel,
        out_type=tuple(outs) if len(outs) > 1 else outs[0],
        mesh=_mesh(),
        compiler_params=_SC_PARAMS,
        scratch_types=[
            pltpu.VMEM((1, CH), jnp.int32),
            pltpu.VMEM((1, CH), jnp.int32),
            pltpu.VMEM((2, CH), jnp.int32),
            pltpu.VMEM((CH, H), jnp.float32),
            pltpu.VMEM((CH, H), jnp.float32),
            pltpu.VMEM((ZB, H), jnp.float32),
            pltpu.VMEM((ZB, 16), jnp.float32),
            pltpu.VMEM_SHARED((ACC + 8, H), jnp.float32),
            pltpu.SemaphoreType.DMA,
        ],
    )
    def k(*refs):
        pos = 0
        dom_h = refs[pos]; pos += 1
        if do_g:
            gsrc_h = refs[pos]; pos += 1
            cee_h = refs[pos]; pos += 1
        if do_l:
            lsrc_h = refs[pos]; pos += 1
        if not do_cnt:
            rcp_h = refs[pos]; pos += 1
        if do_g:
            g_h = refs[pos]; pos += 1
            mg_h = refs[pos]; pos += 1
        if do_l:
            ml_h = refs[pos]; pos += 1
        if do_cnt:
            rcp_h = refs[pos]; pos += 1
        (domv, ceev, idxb, vbuf, ones128, zbuf, rcpb, sums,
         sem) = refs[pos:pos + 9]

        cid = lax.axis_index("c")
        sid = lax.axis_index("s")
        zv = jnp.zeros((16,), jnp.float32)
        ov = jnp.ones((16,), jnp.float32)

        def initrow(r, _):
            for j in range(H // 16):
                zbuf[r, pl.ds(16 * j, 16)] = zv
            return 0

        lax.fori_loop(0, ZB, initrow, 0)

        def onesrow(r, _):
            for j in range(H // 16):
                ones128[r, pl.ds(16 * j, 16)] = ov
            return 0

        lax.fori_loop(0, CH, onesrow, 0)

        def zero_acc():
            for b in range(4):
                pltpu.sync_copy(zbuf, sums.at[pl.ds(sid * STR + b * ZB, ZB)])

            @pl.when(sid == 0)
            def _():
                pltpu.sync_copy(zbuf.at[pl.ds(0, 8)], sums.at[pl.ds(ACC, 8)])

        def build_idx(half):
            dbase = cid * SEG + half * SEGH
            for i5 in range(CH // 16):
                d16 = domv[0, pl.ds(16 * i5, 16)]
                dl = d16 - dbase
                ok = (dl >= 0) & (dl < SEGH)
                idxb[0, pl.ds(16 * i5, 16)] = jnp.where(ok, dl, ACC)

        def accumulate(phase, half):
            def body(kk, _):
                j = sid * NPT + kk
                pltpu.sync_copy(dom_h.at[j], domv)
                if phase == 0:
                    pltpu.sync_copy(cee_h.at[j], ceev)
                    pltpu.async_copy(gsrc_h.at[ceev.at[0]], vbuf,
                                     sem).wait()

                    @pl.when((j % 2) == cid)
                    def _():
                        pltpu.sync_copy(vbuf, g_h.at[pl.ds(j * CH, CH)])
                elif phase == 1:
                    pltpu.sync_copy(lsrc_h.at[pl.ds(j * CH, CH)], vbuf)
                build_idx(half)
                if phase == 2:
                    pltpu.sync_copy(ones128, sums.at[idxb.at[0]], add=True)
                else:
                    pltpu.sync_copy(vbuf, sums.at[idxb.at[0]], add=True)
                return 0

            lax.fori_loop(0, NPT, body, 0)

        def out_row0(half, b):
            return cid * SEGP + half * SEGH + sid * STR + b * ZB

        def finalize_counts(half):
            for b in range(4):
                r0 = sid * STR + b * ZB
                pltpu.sync_copy(sums.at[pl.ds(r0, ZB)], zbuf)

                def frow(r, _):
                    c16 = zbuf[r, pl.ds(0, 16)]
                    rcpb[r, pl.ds(0, 16)] = 1.0 / jnp.maximum(c16, 1.0)
                    return 0

                lax.fori_loop(0, ZB, frow, 0)
                pltpu.sync_copy(rcpb, rcp_h.at[pl.ds(out_row0(half, b), ZB)])
            lax.fori_loop(0, ZB, initrow, 0)

        def finalize(m_h, half):
            for b in range(4):
                r0 = sid * STR + b * ZB
                pltpu.sync_copy(sums.at[pl.ds(r0, ZB)], zbuf)
                pltpu.sync_copy(rcp_h.at[pl.ds(out_row0(half, b), ZB)], rcpb)

                def frow(r, _):
                    rcp = rcpb[r, pl.ds(0, 16)]
                    for j in range(H // 16):
                        zbuf[r, pl.ds(16 * j, 16)] = \
                            zbuf[r, pl.ds(16 * j, 16)] * rcp
                    return 0

                lax.fori_loop(0, ZB, frow, 0)
                pltpu.sync_copy(zbuf, m_h.at[pl.ds(out_row0(half, b), ZB)])
            lax.fori_loop(0, ZB, initrow, 0)

        plan = []
        if do_cnt:
            plan += [(2, None)]
        if do_g:
            plan += [(0, mg_h)]
        if do_l:
            plan += [(1, ml_h)]
        first = True
        for phase, m_h in plan:
            for half in (0, 1):
                if not first:
                    plsc.subcore_barrier()
                zero_acc()
                plsc.subcore_barrier()
                accumulate(phase, half)
                plsc.subcore_barrier()
                if phase == 2:
                    finalize_counts(half)
                else:
                    finalize(m_h, half)
                first = False

    return k(*ins)


def _sc_bcast(dom3d, means_list, C, T2):
    NCHK = T2 // CH
    NPW = NCHK // 32
    SEG, PAD = 10000, 240
    nm = len(means_list)

    @functools.partial(
        pl.kernel,
        out_type=tuple(_f32((T2, H)) for _ in range(nm)) if nm > 1
        else _f32((T2, H)),
        mesh=_mesh(),
        compiler_params=_SC_PARAMS,
        scratch_types=[
            pltpu.VMEM((1, CH), jnp.int32),
            pltpu.VMEM((2, CH), jnp.int32),
            pltpu.VMEM((CH, H), jnp.float32),
            pltpu.SemaphoreType.DMA,
        ],
    )
    def k(*refs):
        dom_h = refs[0]
        m_hs = refs[1:1 + nm]
        o_hs = refs[1 + nm:1 + 2 * nm]
        domv, idxb, vbuf, sem = refs[1 + 2 * nm:]
        cid = lax.axis_index("c")
        sid = lax.axis_index("s")
        w = sid * 2 + cid

        def body(kk, _):
            j = w * NPW + kk
            pltpu.sync_copy(dom_h.at[j], domv)
            for i5 in range(CH // 16):
                d16 = domv[0, pl.ds(16 * i5, 16)]
                idxb[0, pl.ds(16 * i5, 16)] = \
                    d16 + jnp.where(d16 >= SEG, PAD, 0)
            for mi in range(nm):
                pltpu.async_copy(m_hs[mi].at[idxb.at[0]], vbuf, sem).wait()
                pltpu.sync_copy(vbuf, o_hs[mi].at[pl.ds(j * CH, CH)])
            return 0

        lax.fori_loop(0, NPW, body, 0)

    return k(dom3d, *means_list)


def _sc_val(dom3d, h2p, mh2, scale16, C, T2):
    NCHK = T2 // CH
    NPW = NCHK // 32
    SEG, PAD = 10000, 240

    @functools.partial(
        pl.kernel,
        out_type=_f32((T2, H)),
        mesh=_mesh(),
        compiler_params=_SC_PARAMS,
        scratch_types=[
            pltpu.VMEM((1, CH), jnp.int32),
            pltpu.VMEM((2, CH), jnp.int32),
            pltpu.VMEM((CH, H), jnp.float32),
            pltpu.VMEM((CH, H), jnp.float32),
            pltpu.VMEM((16,), jnp.float32),
            pltpu.SemaphoreType.DMA,
        ],
    )
    def k(dom_h, h2_h, m_h, sc_h, out_h, domv, idxb, b1, b2, scv, sem):
        cid = lax.axis_index("c")
        sid = lax.axis_index("s")
        w = sid * 2 + cid
        pltpu.sync_copy(sc_h, scv)
        ev = scv[pl.ds(0, 16)]

        def body(kk, _):
            j = w * NPW + kk
            pltpu.sync_copy(dom_h.at[j], domv)
            pltpu.sync_copy(h2_h.at[pl.ds(j * CH, CH)], b1)
            for i5 in range(CH // 16):
                d16 = domv[0, pl.ds(16 * i5, 16)]
                idxb[0, pl.ds(16 * i5, 16)] = \
                    d16 + jnp.where(d16 >= SEG, PAD, 0)
            pltpu.async_copy(m_h.at[idxb.at[0]], b2, sem).wait()

            def crow(r, _):
                for j8 in range(H // 16):
                    sl = pl.ds(16 * j8, 16)
                    b1[r, sl] = b1[r, sl] * ev + b2[r, sl]
                return 0

            lax.fori_loop(0, CH, crow, 0)
            pltpu.sync_copy(b1, out_h.at[pl.ds(j * CH, CH)])
            return 0

        lax.fori_loop(0, NPW, body, 0)

    return k(dom3d, h2p, mh2, scale16)


def _sc_scatter_edges(val, cee_flat, E2, T2):
    RNG = 8192
    NRANGE = E2 // RNG
    NPSC = NRANGE // 2
    TPT = T2 // 16
    NIT = TPT // 16
    ACC = 8320
    DUMMY = RNG + 8
    STR = ACC // 16
    ZB = STR // 5
    LSZ = TPT + 96

    @functools.partial(
        pl.kernel,
        out_type=_f32((E2, H)),
        mesh=_mesh(),
        compiler_params=_SC_PARAMS,
        scratch_types=[
            pltpu.VMEM((TPT,), jnp.int32),
            pltpu.VMEM((LSZ,), jnp.int32),
            pltpu.VMEM((LSZ,), jnp.int32),
            pltpu.VMEM((LSZ // CH + 1, CH), jnp.int32),
            pltpu.VMEM((CH, H), jnp.float32),
            pltpu.VMEM((ZB, H), jnp.float32),
            pltpu.VMEM_SHARED((ACC, H), jnp.float32),
            pltpu.SemaphoreType.DMA,
        ],
    )
    def k(val_h, cee_h, out_h, ceebuf, elist, tlist, e2d, vbuf, zbuf, acc,
          sem):
        cid = lax.axis_index("c")
        sid = lax.axis_index("s")
        zv = jnp.zeros((16,), jnp.float32)
        iota = lax.iota(jnp.int32, 16)

        def zrow(r, _):
            for j in range(H // 16):
                zbuf[r, pl.ds(16 * j, 16)] = zv
            return 0

        lax.fori_loop(0, ZB, zrow, 0)
        tb = sid * TPT
        pltpu.sync_copy(cee_h.at[pl.ds(tb, TPT)], ceebuf)

        def one_pass(p, _):
            base = (cid * NPSC + p) * RNG
            for b in range(5):
                pltpu.sync_copy(zbuf, acc.at[pl.ds(sid * STR + b * ZB, ZB)])
            plsc.subcore_barrier()

            def scan(i, m):
                ev16 = ceebuf[pl.ds(16 * i, 16)]
                el = ev16 - base
                ok = (el >= 0) & (el < RNG)
                c16 = plsc.all_reduce_population_count(ok)
                plsc.store_compressed(elist.at[pl.ds(m, 16)],
                                      jnp.where(ok, el, 0), mask=ok)
                plsc.store_compressed(tlist.at[pl.ds(m, 16)],
                                      tb + 16 * i + iota, mask=ok)
                return m + c16[0]

            m = lax.fori_loop(0, NIT, scan, jnp.int32(0))
            for g5 in range(5):
                elist[pl.ds(m + 16 * g5, 16)] = jnp.full((16,), DUMMY,
                                                         jnp.int32)
                tlist[pl.ds(m + 16 * g5, 16)] = jnp.zeros((16,), jnp.int32)
            nch = (m + CH - 1) // CH

            def copy2d(ch2, _):
                for i5 in range(CH // 16):
                    e2d[ch2, pl.ds(16 * i5, 16)] = \
                        elist[pl.ds(CH * ch2 + 16 * i5, 16)]
                return 0

            lax.fori_loop(0, nch, copy2d, 0)

            def gsc(ch2, _):
                pltpu.async_copy(val_h.at[tlist.at[pl.ds(CH * ch2, CH)]],
                                 vbuf, sem).wait()
                pltpu.sync_copy(vbuf, acc.at[e2d.at[ch2]], add=True)
                return 0

            lax.fori_loop(0, nch, gsc, 0)
            plsc.subcore_barrier()
            wr = RNG // 16
            pltpu.sync_copy(acc.at[pl.ds(sid * wr, wr)],
                            out_h.at[pl.ds(base + sid * wr, wr)])
            plsc.subcore_barrier()
            return 0

        lax.fori_loop(0, NPSC, one_pass, 0)

    return k(val, cee_flat)


def _tc_fused(R, BR, ins, stats, weights, epsmat, fn, outs_spec, name):
    nb = R // BR
    n_in, n_st, n_w = len(ins), len(stats), len(weights)

    def body(*refs):
        i = pl.program_id(0)
        in_refs = refs[:n_in]
        st_refs = refs[n_in:n_in + n_st]
        w_refs = refs[n_in + n_st:n_in + n_st + n_w]
        eps_ref = refs[n_in + n_st + n_w]
        rest = refs[n_in + n_st + n_w + 1:]
        n_o = len(outs_spec) + sum(1 for _, ws in outs_spec if ws)
        out_refs = rest[:n_o]
        scr_refs = rest[n_o:]

        @pl.when(i == 0)
        def _():
            for st_ref, scr in zip(st_refs, scr_refs):
                s = jnp.sum(st_ref[...], axis=0)
                mu = s[0:1] / R
                var = s[1:2] / R - mu * mu
                rs = lax.rsqrt(var + EPS)
                scr[0:1, :] = mu
                scr[1:2, :] = rs

        finstats = [(scr[0:1, :], scr[1:2, :]) for scr in scr_refs]
        outs = fn([r[...] for r in in_refs], finstats,
                  [r[...] for r in w_refs], eps_ref)
        oi = 0
        for o, (ko, ws) in zip(outs, outs_spec):
            out_refs[oi][...] = o
            oi += 1
            if ws:
                out_refs[oi][0, 0, :] = jnp.sum(o, axis=0)
                out_refs[oi][0, 1, :] = jnp.sum(o * o, axis=0)
                oi += 1

    in_specs = (
        [pl.BlockSpec((BR, a.shape[1]), lambda i: (i, 0)) for a in ins]
        + [pl.BlockSpec(p.shape, lambda i: (0, 0, 0)) for p in stats]
        + [pl.BlockSpec(w.shape, lambda i: (0, 0)) for w in weights]
        + [pl.BlockSpec(epsmat.shape, lambda i: (0, 0))]
    )
    out_shape, out_specs = [], []
    for ko, ws in outs_spec:
        out_shape.append(_f32((R, ko)))
        out_specs.append(pl.BlockSpec((BR, ko), lambda i: (i, 0)))
        if ws:
            out_shape.append(_f32((nb, 2, ko)))
            out_specs.append(pl.BlockSpec((1, 2, ko), lambda i: (i, 0, 0)))
    scratch = [pltpu.VMEM((2, p.shape[2]), jnp.float32) for p in stats]
    return pl.pallas_call(
        body,
        grid=(nb,),
        in_specs=in_specs,
        out_specs=out_specs,
        out_shape=out_shape,
        scratch_shapes=scratch,
        name=name,
    )(*ins, *stats, *weights, epsmat)


def _nrm(y, st):
    mu, rs = st
    return jnp.maximum((y - mu) * rs, 0.0)


def _mm(x, w):
    return lax.dot_general(x, w, (((1,), (1,)), ((), ())),
                           preferred_element_type=jnp.float32)


def _tc_node(node_rep, partials, epsmat, Wa, Wb, N):
    def body(x_ref, p_ref, eps_ref, wa_ref, wb_ref, o_ref):
        ev = eps_ref[0:1, :]
        p = p_ref[...]
        n = x_ref.shape[0]
        x = x_ref[...] * ev + p[0, :n] + p[1, :n]
        y1 = _mm(x, wa_ref[...])
        mu = jnp.mean(y1, axis=0, keepdims=True)
        var = jnp.mean((y1 - mu) ** 2, axis=0, keepdims=True)
        h = jnp.maximum((y1 - mu) * lax.rsqrt(var + EPS), 0.0)
        y2 = _mm(h, wb_ref[...])
        mu2 = jnp.mean(y2, axis=0, keepdims=True)
        var2 = jnp.mean((y2 - mu2) ** 2, axis=0, keepdims=True)
        o_ref[...] = jnp.maximum((y2 - mu2) * lax.rsqrt(var2 + EPS), 0.0)

    return pl.pallas_call(
        body,
        out_shape=_f32((N, H)),
        name="node_mlp2",
    )(node_rep, partials, epsmat, Wa, Wb)


def kernel(node_rep, edge_rep, cycle_rep, edge_index, cycle_entry_edge,
           cycle_domain, W_ne_lift1, W_ne_lift2, W_ne_lvl1, W_ne_lvl2a,
           W_ne_lvl2b, eps_ne_1, eps_ne_2, W_ec_lift1, W_ec_lift2, W_ec_lvl1,
           W_ec_lvl2a, W_ec_lvl2b, eps_ec_11, eps_ec_12, eps_ec_2, W_mlp):
    N = node_rep.shape[0]
    E = edge_rep.shape[0]
    T = cycle_rep.shape[0]
    C = 20000
    BR = 1000
    T2 = 32 * CH * 47
    E2 = 40 * 8192

    ei = edge_index.astype(jnp.int32)
    CPW = E // (32 * CH)
    i0_3d = ei[0].reshape(32, CPW, CH)
    i1_3d = ei[1].reshape(32, CPW, CH)
    cee = cycle_entry_edge.astype(jnp.int32)
    dom = cycle_domain.astype(jnp.int32)
    cee_p = jnp.concatenate([cee, jnp.full((T2 - T,), E, jnp.int32)])
    dom_p = jnp.concatenate([dom, jnp.full((T2 - T,), C, jnp.int32)])
    cee3d = cee_p.reshape(T2 // CH, 1, CH)
    dom3d = dom_p.reshape(T2 // CH, 1, CH)
    crep_p = jnp.concatenate(
        [cycle_rep, jnp.zeros((T2 - T, H), jnp.float32)], axis=0)

    epsmat = jnp.broadcast_to(
        jnp.stack([1.0 + eps_ne_1, 1.0 + eps_ne_2, 1.0 + eps_ec_11,
                   1.0 + eps_ec_12, 1.0 + eps_ec_2,
                   jnp.float32(0), jnp.float32(0), jnp.float32(0)])[:, None],
        (8, H))
    eps12_16 = jnp.broadcast_to((1.0 + eps_ec_12)[None], (16,))

    g0, g1 = _sc_gather2(node_rep, i0_3d, i1_3d, E)

    def fn_s1(xs, fs, ws, eps_ref):
        g0b, g1b, eb = xs
        la = g0b + g1b
        y1e = _mm(jnp.concatenate([la, eb], axis=1), ws[0])
        x2 = eb * eps_ref[1:2, :] + la
        y1o = _mm(x2, ws[1])
        return [y1e, y1o]

    y1e, p1e, y1o, p1o = _tc_fused(
        E, BR, [g0, g1, edge_rep], [], [W_ne_lvl1, W_ne_lift1], epsmat,
        fn_s1, [(H, True), (2 * H, True)], "s1_edge_lin")

    def fn_s2(xs, fs, ws, eps_ref):
        h1 = _nrm(xs[0], fs[0])
        y2o = _mm(_nrm(xs[1], fs[1]), ws[0])
        return [h1, y2o]

    h1, y2o, p2o = _tc_fused(
        E, BR, [y1e, y1o], [p1e, p1o], [W_ne_lift2], epsmat,
        fn_s2, [(H, False), (H, True)], "s2_edge_lin")

    nparts = _sc_scatter_nodes(h1, i0_3d, i1_3d, E, N)
    node_out = _tc_node(node_rep, nparts, epsmat, W_ne_lvl2a, W_ne_lvl2b, N)

    g_pad, mg, mc, rcp = _sc_seg_means(dom3d, C, T2, gather_src=edge_rep,
                                       cee3d=cee3d, linear_src=crep_p)
    gm_pad, cycb_pad = _sc_bcast(dom3d, [mg, mc], C, T2)

    def fn_s5(xs, fs, ws, eps_ref):
        gb, gmb, cb, cbb = xs
        y1c = _mm(jnp.concatenate([gb, gmb, cb], axis=1), ws[0])
        ev = eps_ref[4:5, :]
        x2 = jnp.concatenate([cb * ev + gb, cbb * ev + gmb], axis=1)
        y1k = _mm(x2, ws[1])
        return [y1c, y1k]

    y1c, p1c, y1k, p1k = _tc_fused(
        T, BR, [g_pad, gm_pad, cycle_rep, cycb_pad], [],
        [W_ec_lvl1, W_ec_lift1], epsmat,
        fn_s5, [(H, True), (2 * H, True)], "s5_cyc_lin")

    def fn_s6(xs, fs, ws, eps_ref):
        h2 = _nrm(xs[0], fs[0])
        y2k = _mm(_nrm(xs[1], fs[1]), ws[0])
        return [h2, y2k]

    h2, y2k, p2k = _tc_fused(
        T, BR, [y1c, y1k], [p1c, p1k], [W_ec_lift2], epsmat,
        fn_s6, [(H, False), (H, True)], "s6_cyc_lin")

    def fn_norm_only(xs, fs, ws, eps_ref):
        return [_nrm(xs[0], fs[0])]

    cycle_out, = _tc_fused(T, BR, [y2k], [p2k], [], epsmat,
                           fn_norm_only, [(H, False)], "s9_cyc_out")

    h2p = jnp.concatenate([h2, jnp.zeros((T2 - T, H), jnp.float32)], axis=0)
    mh2 = _sc_seg_means(dom3d, C, T2, linear_src=h2p, rcp_in=rcp)
    val = _sc_val(dom3d, h2p, mh2, eps12_16, C, T2)
    lvlc = _sc_scatter_edges(val, cee_p, E2, T2)[:E]

    def fn_s10(xs, fs, ws, eps_ref):
        x = xs[0] * eps_ref[2:3, :] + xs[1]
        return [_mm(x, ws[0])]

    y1f, p1f = _tc_fused(E, BR, [edge_rep, lvlc], [], [W_ec_lvl2a], epsmat,
                         fn_s10, [(2 * H, True)], "s10_edge2")

    def fn_s11(xs, fs, ws, eps_ref):
        return [_mm(_nrm(xs[0], fs[0]), ws[0])]

    y2f, p2f = _tc_fused(E, BR, [y1f], [p1f], [W_ec_lvl2b], epsmat,
                         fn_s11, [(H, True)], "s11_edge2")

    def fn_s12(xs, fs, ws, eps_ref):
        x = jnp.concatenate([_nrm(xs[0], fs[0]), _nrm(xs[1], fs[1])], axis=1)
        return [_mm(x, ws[0])]

    z, pz = _tc_fused(E, BR, [y2o, y2f], [p2o, p2f], [W_mlp], epsmat,
                      fn_s12, [(H, True)], "s12_final_lin")

    edge_out, = _tc_fused(E, BR, [z], [pz], [], epsmat,
                          fn_norm_only, [(H, False)], "s12b_final_norm")

    return (node_out, edge_out, cycle_out)


# speedup vs baseline: 1.1767x; 1.1767x over previous
import functools

import jax
import jax.numpy as jnp
from jax import lax
from jax.experimental import pallas as pl
from jax.experimental.pallas import tpu as pltpu, tpu_sc as plsc

EPS = 1e-05
H = 128
CH = 80

_SC_PARAMS = pltpu.CompilerParams(needs_layout_passes=False)


def _mesh():
    return plsc.VectorSubcoreMesh(core_axis_name="c", subcore_axis_name="s")


def _f32(shape):
    return jax.ShapeDtypeStruct(shape, jnp.float32)


def _sc_gather2(table, i0_3d, i1_3d, E):
    CPW = E // (32 * CH)

    @functools.partial(
        pl.kernel,
        out_type=(_f32((E, H)), _f32((E, H))),
        mesh=_mesh(),
        compiler_params=_SC_PARAMS,
        scratch_types=[
            pltpu.VMEM((CPW, CH), jnp.int32),
            pltpu.VMEM((CPW, CH), jnp.int32),
            pltpu.VMEM((CH, H), jnp.float32),
            pltpu.VMEM((CH, H), jnp.float32),
            pltpu.SemaphoreType.DMA,
            pltpu.SemaphoreType.DMA,
        ],
    )
    def k(tab_h, i0_h, i1_h, g0_h, g1_h, i0v, i1v, b0, b1, s0, s1):
        cid = lax.axis_index("c")
        sid = lax.axis_index("s")
        w = sid * 2 + cid
        r0 = w * CPW
        pltpu.sync_copy(i0_h.at[w], i0v)
        pltpu.sync_copy(i1_h.at[w], i1v)

        def body(ch, _):
            cp0 = pltpu.async_copy(tab_h.at[i0v.at[ch]], b0, s0)
            cp1 = pltpu.async_copy(tab_h.at[i1v.at[ch]], b1, s1)
            cp0.wait()
            cp1.wait()
            base = (r0 + ch) * CH
            pltpu.sync_copy(b0, g0_h.at[pl.ds(base, CH)])
            pltpu.sync_copy(b1, g1_h.at[pl.ds(base, CH)])
            return 0

        lax.fori_loop(0, CPW, body, 0)

    return k(table, i0_3d, i1_3d)


def _sc_scatter_nodes(h1, i0_3d, i1_3d, E, N):
    CPT = E // (16 * CH)
    NH = N // 2
    ACC = 5120
    STR = 312

    @functools.partial(
        pl.kernel,
        out_type=_f32((N, H)),
        mesh=_mesh(),
        compiler_params=_SC_PARAMS,
        scratch_types=[
            pltpu.VMEM((25, 1, CH), jnp.int32),
            pltpu.VMEM((25, 1, CH), jnp.int32),
            pltpu.VMEM((2, CH, H), jnp.float32),
            pltpu.VMEM((2, CH), jnp.int32),
            pltpu.VMEM((64, H), jnp.float32),
            pltpu.VMEM_SHARED((ACC + 8, H), jnp.float32),
            pltpu.SemaphoreType.DMA,
        ],
    )
    def k(h1_h, i0_h, i1_h, out_h, i0v, i1v, hbuf, idxb, zbuf, nacc, sem):
        cid = lax.axis_index("c")
        sid = lax.axis_index("s")
        zv = jnp.zeros((16,), jnp.float32)

        def zrow(r, _):
            for j in range(H // 16):
                zbuf[r, pl.ds(16 * j, 16)] = zv
            return 0

        lax.fori_loop(0, 64, zrow, 0)
        for b in range(5):
            pltpu.sync_copy(zbuf, nacc.at[pl.ds(sid * 320 + b * 64, 64)])

        @pl.when(sid == 0)
        def _():
            pltpu.sync_copy(zbuf.at[pl.ds(0, 8)], nacc.at[pl.ds(ACC, 8)])

        plsc.subcore_barrier()
        nbase = cid * NH

        cp = pltpu.async_copy(h1_h.at[pl.ds(sid * CPT * CH, CH)],
                              hbuf.at[0], sem)

        def body(ch, _):
            g = ch % 25

            @pl.when(g == 0)
            def _():
                pltpu.sync_copy(i0_h.at[pl.ds(sid * CPT + ch, 25)], i0v)
                pltpu.sync_copy(i1_h.at[pl.ds(sid * CPT + ch, 25)], i1v)

            nxt = (ch + 1) % 2
            cur = ch % 2
            pltpu.make_async_copy(h1_h.at[pl.ds(0, CH)], hbuf.at[cur],
                                  sem).wait()

            @pl.when(ch + 1 < CPT)
            def _():
                base2 = (sid * CPT + ch + 1) * CH
                pltpu.async_copy(h1_h.at[pl.ds(base2, CH)], hbuf.at[nxt],
                                 sem)

            for i5 in range(CH // 16):
                sl = pl.ds(16 * i5, 16)
                v0 = i0v[g, 0, sl] - nbase
                v1 = i1v[g, 0, sl] - nbase
                idxb[0, sl] = jnp.where((v0 >= 0) & (v0 < NH), v0, ACC)
                idxb[1, sl] = jnp.where((v1 >= 0) & (v1 < NH), v1, ACC)
            pltpu.sync_copy(hbuf.at[cur], nacc.at[idxb.at[0]], add=True)
            pltpu.sync_copy(hbuf.at[cur], nacc.at[idxb.at[1]], add=True)
            return 0

        lax.fori_loop(0, CPT, body, 0)
        _ = cp
        plsc.subcore_barrier()
        pltpu.sync_copy(nacc.at[pl.ds(sid * STR, STR)],
                        out_h.at[pl.ds(nbase + sid * STR, STR)])

        @pl.when(sid == 15)
        def _():
            pltpu.sync_copy(nacc.at[pl.ds(4992, 8)],
                            out_h.at[pl.ds(nbase + 4992, 8)])

    return k(h1, i0_3d, i1_3d)


def _sc_seg_means(dom3d, C, T2, gather_src=None, cee3d=None, linear_src=None,
                  rcp_in=None):
    NCHK = T2 // CH
    NPT = NCHK // 16
    SEG = C // 2
    SEGH = SEG // 2
    SEGP = 10240
    ACC = 5120
    STR = ACC // 16
    ZB = STR // 4
    do_g = gather_src is not None
    do_l = linear_src is not None
    do_cnt = rcp_in is None

    outs = []
    if do_g:
        outs.append(_f32((T2, H)))
        outs.append(_f32((2 * SEGP, H)))
    if do_l:
        outs.append(_f32((2 * SEGP, H)))
    if do_cnt:
        outs.append(_f32((2 * SEGP, 16)))

    ins = [dom3d]
    if do_g:
        ins += [gather_src, cee3d]
    if do_l:
        ins += [linear_src]
    if not do_cnt:
        ins += [rcp_in]

    @functools.partial(
        pl.kernel,
        out_type=tuple(outs) if len(outs) > 1 else outs[0],
        mesh=_mesh(),
        compiler_params=_SC_PARAMS,
        scratch_types=[
            pltpu.VMEM((1, CH), jnp.int32),
            pltpu.VMEM((1, CH), jnp.int32),
            pltpu.VMEM((2, CH), jnp.int32),
            pltpu.VMEM((CH, H), jnp.float32),
            pltpu.VMEM((CH, H), jnp.float32),
            pltpu.VMEM((ZB, H), jnp.float32),
            pltpu.VMEM((ZB, 16), jnp.float32),
            pltpu.VMEM_SHARED((ACC + 8, H), jnp.float32),
            pltpu.SemaphoreType.DMA,
        ],
    )
    def k(*refs):
        pos = 0
        dom_h = refs[pos]; pos += 1
        if do_g:
            gsrc_h = refs[pos]; pos += 1
            cee_h = refs[pos]; pos += 1
        if do_l:
            lsrc_h = refs[pos]; pos += 1
        if not do_cnt:
            rcp_h = refs[pos]; pos += 1
        if do_g:
            g_h = refs[pos]; pos += 1
            mg_h = refs[pos]; pos += 1
        if do_l:
            ml_h = refs[pos]; pos += 1
        if do_cnt:
            rcp_h = refs[pos]; pos += 1
        (domv, ceev, idxb, vbuf, ones128, zbuf, rcpb, sums,
         sem) = refs[pos:pos + 9]

        cid = lax.axis_index("c")
        sid = lax.axis_index("s")
        zv = jnp.zeros((16,), jnp.float32)
        ov = jnp.ones((16,), jnp.float32)

        def initrow(r, _):
            for j in range(H // 16):
                zbuf[r, pl.ds(16 * j, 16)] = zv
            return 0

        lax.fori_loop(0, ZB, initrow, 0)

        def onesrow(r, _):
            for j in range(H // 16):
                ones128[r, pl.ds(16 * j, 16)] = ov
            return 0

        lax.fori_loop(0, CH, onesrow, 0)

        def zero_acc():
            for b in range(4):
                pltpu.sync_copy(zbuf, sums.at[pl.ds(sid * STR + b * ZB, ZB)])

            @pl.when(sid == 0)
            def _():
                pltpu.sync_copy(zbuf.at[pl.ds(0, 8)], sums.at[pl.ds(ACC, 8)])

        def build_idx(half):
            dbase = cid * SEG + half * SEGH
            for i5 in range(CH // 16):
                d16 = domv[0, pl.ds(16 * i5, 16)]
                dl = d16 - dbase
                ok = (dl >= 0) & (dl < SEGH)
                idxb[0, pl.ds(16 * i5, 16)] = jnp.where(ok, dl, ACC)

        def accumulate(phase, half):
            def body(kk, _):
                j = sid * NPT + kk
                pltpu.sync_copy(dom_h.at[j], domv)
                if phase == 0:
                    pltpu.sync_copy(cee_h.at[j], ceev)
                    pltpu.async_copy(gsrc_h.at[ceev.at[0]], vbuf,
                                     sem).wait()

                    @pl.when((j % 2) == cid)
                    def _():
                        pltpu.sync_copy(vbuf, g_h.at[pl.ds(j * CH, CH)])
                elif phase == 1:
                    pltpu.sync_copy(lsrc_h.at[pl.ds(j * CH, CH)], vbuf)
                build_idx(half)
                if phase == 2:
                    pltpu.sync_copy(ones128, sums.at[idxb.at[0]], add=True)
                else:
                    pltpu.sync_copy(vbuf, sums.at[idxb.at[0]], add=True)
                return 0

            lax.fori_loop(0, NPT, body, 0)

        def out_row0(half, b):
            return cid * SEGP + half * SEGH + sid * STR + b * ZB

        def finalize_counts(half):
            for b in range(4):
                r0 = sid * STR + b * ZB
                pltpu.sync_copy(sums.at[pl.ds(r0, ZB)], zbuf)

                def frow(r, _):
                    c16 = zbuf[r, pl.ds(0, 16)]
                    rcpb[r, pl.ds(0, 16)] = 1.0 / jnp.maximum(c16, 1.0)
                    return 0

                lax.fori_loop(0, ZB, frow, 0)
                pltpu.sync_copy(rcpb, rcp_h.at[pl.ds(out_row0(half, b), ZB)])
            lax.fori_loop(0, ZB, initrow, 0)

        def finalize(m_h, half):
            for b in range(4):
                r0 = sid * STR + b * ZB
                pltpu.sync_copy(sums.at[pl.ds(r0, ZB)], zbuf)
                pltpu.sync_copy(rcp_h.at[pl.ds(out_row0(half, b), ZB)], rcpb)

                def frow(r, _):
                    rcp = rcpb[r, pl.ds(0, 16)]
                    for j in range(H // 16):
                        zbuf[r, pl.ds(16 * j, 16)] = \
                            zbuf[r, pl.ds(16 * j, 16)] * rcp
                    return 0

                lax.fori_loop(0, ZB, frow, 0)
                pltpu.sync_copy(zbuf, m_h.at[pl.ds(out_row0(half, b), ZB)])
            lax.fori_loop(0, ZB, initrow, 0)

        plan = []
        if do_cnt:
            plan += [(2, None)]
        if do_g:
            plan += [(0, mg_h)]
        if do_l:
            plan += [(1, ml_h)]
        first = True
        for phase, m_h in plan:
            for half in (0, 1):
                if not first:
                    plsc.subcore_barrier()
                zero_acc()
                plsc.subcore_barrier()
                accumulate(phase, half)
                plsc.subcore_barrier()
                if phase == 2:
                    finalize_counts(half)
                else:
                    finalize(m_h, half)
                first = False

    return k(*ins)


def _sc_bcast(dom3d, means_list, C, T2):
    NCHK = T2 // CH
    NPW = NCHK // 32
    SEG, PAD = 10000, 240
    nm = len(means_list)

    @functools.partial(
        pl.kernel,
        out_type=tuple(_f32((T2, H)) for _ in range(nm)) if nm > 1
        else _f32((T2, H)),
        mesh=_mesh(),
        compiler_params=_SC_PARAMS,
        scratch_types=[
            pltpu.VMEM((1, CH), jnp.int32),
            pltpu.VMEM((2, CH), jnp.int32),
            pltpu.VMEM((CH, H), jnp.float32),
            pltpu.SemaphoreType.DMA,
        ],
    )
    def k(*refs):
        dom_h = refs[0]
        m_hs = refs[1:1 + nm]
        o_hs = refs[1 + nm:1 + 2 * nm]
        domv, idxb, vbuf, sem = refs[1 + 2 * nm:]
        cid = lax.axis_index("c")
        sid = lax.axis_index("s")
        w = sid * 2 + cid

        def body(kk, _):
            j = w * NPW + kk
            pltpu.sync_copy(dom_h.at[j], domv)
            for i5 in range(CH // 16):
                d16 = domv[0, pl.ds(16 * i5, 16)]
                idxb[0, pl.ds(16 * i5, 16)] = \
                    d16 + jnp.where(d16 >= SEG, PAD, 0)
            for mi in range(nm):
                pltpu.async_copy(m_hs[mi].at[idxb.at[0]], vbuf, sem).wait()
                pltpu.sync_copy(vbuf, o_hs[mi].at[pl.ds(j * CH, CH)])
            return 0

        lax.fori_loop(0, NPW, body, 0)

    return k(dom3d, *means_list)


def _sc_val(dom3d, h2p, mh2, scale16, C, T2):
    NCHK = T2 // CH
    NPW = NCHK // 32
    SEG, PAD = 10000, 240

    @functools.partial(
        pl.kernel,
        out_type=_f32((T2, H)),
        mesh=_mesh(),
        compiler_params=_SC_PARAMS,
        scratch_types=[
            pltpu.VMEM((1, CH), jnp.int32),
            pltpu.VMEM((2, CH), jnp.int32),
            pltpu.VMEM((CH, H), jnp.float32),
            pltpu.VMEM((CH, H), jnp.float32),
            pltpu.VMEM((16,), jnp.float32),
            pltpu.SemaphoreType.DMA,
        ],
    )
    def k(dom_h, h2_h, m_h, sc_h, out_h, domv, idxb, b1, b2, scv, sem):
        cid = lax.axis_index("c")
        sid = lax.axis_index("s")
        w = sid * 2 + cid
        pltpu.sync_copy(sc_h, scv)
        ev = scv[pl.ds(0, 16)]

        def body(kk, _):
            j = w * NPW + kk
            pltpu.sync_copy(dom_h.at[j], domv)
            pltpu.sync_copy(h2_h.at[pl.ds(j * CH, CH)], b1)
            for i5 in range(CH // 16):
                d16 = domv[0, pl.ds(16 * i5, 16)]
                idxb[0, pl.ds(16 * i5, 16)] = \
                    d16 + jnp.where(d16 >= SEG, PAD, 0)
            pltpu.async_copy(m_h.at[idxb.at[0]], b2, sem).wait()

            def crow(r, _):
                for j8 in range(H // 16):
                    sl = pl.ds(16 * j8, 16)
                    b1[r, sl] = b1[r, sl] * ev + b2[r, sl]
                return 0

            lax.fori_loop(0, CH, crow, 0)
            pltpu.sync_copy(b1, out_h.at[pl.ds(j * CH, CH)])
            return 0

        lax.fori_loop(0, NPW, body, 0)

    return k(dom3d, h2p, mh2, scale16)


def _sc_scatter_edges(val, cee_flat, E2, T2):
    RNG = 8192
    NPSC = E2 // RNG // 2
    TPT = T2 // 16
    NIT = TPT // 16
    ACC = 8320
    DUMMY = RNG + 8
    GCH = 128
    LSZ = TPT + 2 * GCH

    @functools.partial(
        pl.kernel,
        out_type=_f32((E2, H)),
        mesh=_mesh(),
        compiler_params=_SC_PARAMS,
        scratch_types=[
            pltpu.VMEM((TPT,), jnp.int32),
            pltpu.VMEM((LSZ,), jnp.int32),
            pltpu.VMEM((LSZ // GCH + 1, GCH), jnp.int32),
            pltpu.VMEM((GCH, H), jnp.float32),
            pltpu.VMEM((GCH, H), jnp.float32),
            pltpu.VMEM_SHARED((ACC, H), jnp.float32),
            pltpu.SemaphoreType.DMA,
        ],
    )
    def k(val_h, cee_h, out_h, ceebuf, tlist, e2d, vbuf, zbuf, acc, sem):
        cid = lax.axis_index("c")
        sid = lax.axis_index("s")
        zv = jnp.zeros((16,), jnp.float32)
        iota = lax.iota(jnp.int32, 16)

        def zrow(r, _):
            for j in range(H // 16):
                zbuf[r, pl.ds(16 * j, 16)] = zv
            return 0

        lax.fori_loop(0, GCH, zrow, 0)
        tb = sid * TPT
        pltpu.sync_copy(cee_h.at[pl.ds(tb, TPT)], ceebuf)
        for b in range(4):
            pltpu.sync_copy(zbuf, acc.at[pl.ds(sid * 520 + b * GCH, GCH)])
        pltpu.sync_copy(zbuf.at[pl.ds(0, 8)],
                        acc.at[pl.ds(sid * 520 + 512, 8)])
        plsc.subcore_barrier()

        def one_pass(p, _):
            base = (cid * NPSC + p) * RNG

            def scan(i, m):
                ev16 = ceebuf[pl.ds(16 * i, 16)]
                el = ev16 - base
                ok = (el >= 0) & (el < RNG)
                c16 = plsc.all_reduce_population_count(ok)
                plsc.store_compressed(tlist.at[pl.ds(m, 16)],
                                      16 * i + iota, mask=ok)
                return m + c16[0]

            m = lax.fori_loop(0, NIT, scan, jnp.int32(0))
            for g5 in range(GCH // 16):
                tlist[pl.ds(m + 16 * g5, 16)] = jnp.zeros((16,), jnp.int32)
            nch = (m + GCH - 1) // GCH

            def copy2d(ch2, _):
                for i5 in range(GCH // 16):
                    pos = GCH * ch2 + 16 * i5
                    tloc = tlist[pl.ds(pos, 16)]
                    ee = plsc.load_gather(ceebuf, [tloc]) - base
                    valid = (pos + iota) < m
                    e2d[ch2, pl.ds(16 * i5, 16)] = \
                        jnp.where(valid, ee, DUMMY)
                    tlist[pl.ds(pos, 16)] = tloc + tb
                return 0

            lax.fori_loop(0, nch, copy2d, 0)

            def gsc(ch2, _):
                pltpu.async_copy(
                    val_h.at[tlist.at[pl.ds(GCH * ch2, GCH)]],
                    vbuf, sem).wait()
                pltpu.sync_copy(vbuf, acc.at[e2d.at[ch2]], add=True)
                return 0

            lax.fori_loop(0, nch, gsc, 0)
            plsc.subcore_barrier()
            wr = RNG // 16
            pltpu.sync_copy(acc.at[pl.ds(sid * wr, wr)],
                            out_h.at[pl.ds(base + sid * wr, wr)])
            plsc.subcore_barrier()

            def tz(ch2, _):
                pltpu.sync_copy(zbuf, acc.at[e2d.at[ch2]])
                return 0

            lax.fori_loop(0, nch, tz, 0)
            plsc.subcore_barrier()
            return 0

        lax.fori_loop(0, NPSC, one_pass, 0)

    return k(val, cee_flat)


def _tc_fused(R, BR, ins, stats, weights, epsmat, fn, outs_spec, name):
    nb = R // BR
    n_in, n_st, n_w = len(ins), len(stats), len(weights)

    def body(*refs):
        i = pl.program_id(0)
        in_refs = refs[:n_in]
        st_refs = refs[n_in:n_in + n_st]
        w_refs = refs[n_in + n_st:n_in + n_st + n_w]
        eps_ref = refs[n_in + n_st + n_w]
        rest = refs[n_in + n_st + n_w + 1:]
        n_o = len(outs_spec) + sum(1 for _, ws in outs_spec if ws)
        out_refs = rest[:n_o]
        scr_refs = rest[n_o:]

        @pl.when(i == 0)
        def _():
            for st_ref, scr in zip(st_refs, scr_refs):
                s = jnp.sum(st_ref[...], axis=0)
                mu = s[0:1] / R
                var = s[1:2] / R - mu * mu
                rs = lax.rsqrt(var + EPS)
                scr[0:1, :] = mu
                scr[1:2, :] = rs

        finstats = [(scr[0:1, :], scr[1:2, :]) for scr in scr_refs]
        outs = fn([r[...] for r in in_refs], finstats,
                  [r[...] for r in w_refs], eps_ref)
        oi = 0
        for o, (ko, ws) in zip(outs, outs_spec):
            out_refs[oi][...] = o
            oi += 1
            if ws:
                out_refs[oi][0, 0, :] = jnp.sum(o, axis=0)
                out_refs[oi][0, 1, :] = jnp.sum(o * o, axis=0)
                oi += 1

    in_specs = (
        [pl.BlockSpec((BR, a.shape[1]), lambda i: (i, 0)) for a in ins]
        + [pl.BlockSpec(p.shape, lambda i: (0, 0, 0)) for p in stats]
        + [pl.BlockSpec(w.shape, lambda i: (0, 0)) for w in weights]
        + [pl.BlockSpec(epsmat.shape, lambda i: (0, 0))]
    )
    out_shape, out_specs = [], []
    for ko, ws in outs_spec:
        out_shape.append(_f32((R, ko)))
        out_specs.append(pl.BlockSpec((BR, ko), lambda i: (i, 0)))
        if ws:
            out_shape.append(_f32((nb, 2, ko)))
            out_specs.append(pl.BlockSpec((1, 2, ko), lambda i: (i, 0, 0)))
    scratch = [pltpu.VMEM((2, p.shape[2]), jnp.float32) for p in stats]
    return pl.pallas_call(
        body,
        grid=(nb,),
        in_specs=in_specs,
        out_specs=out_specs,
        out_shape=out_shape,
        scratch_shapes=scratch,
        name=name,
    )(*ins, *stats, *weights, epsmat)


def _nrm(y, st):
    mu, rs = st
    return jnp.maximum((y - mu) * rs, 0.0)


def _mm(x, w):
    return lax.dot_general(x, w, (((1,), (1,)), ((), ())),
                           preferred_element_type=jnp.float32)


def _tc_node(node_rep, partials, epsmat, Wa, Wb, N):
    def body(x_ref, p_ref, eps_ref, wa_ref, wb_ref, o_ref):
        ev = eps_ref[0:1, :]
        x = x_ref[...] * ev + p_ref[...]
        y1 = _mm(x, wa_ref[...])
        mu = jnp.mean(y1, axis=0, keepdims=True)
        var = jnp.mean((y1 - mu) ** 2, axis=0, keepdims=True)
        h = jnp.maximum((y1 - mu) * lax.rsqrt(var + EPS), 0.0)
        y2 = _mm(h, wb_ref[...])
        mu2 = jnp.mean(y2, axis=0, keepdims=True)
        var2 = jnp.mean((y2 - mu2) ** 2, axis=0, keepdims=True)
        o_ref[...] = jnp.maximum((y2 - mu2) * lax.rsqrt(var2 + EPS), 0.0)

    return pl.pallas_call(
        body,
        out_shape=_f32((N, H)),
        name="node_mlp2",
    )(node_rep, partials, epsmat, Wa, Wb)


def kernel(node_rep, edge_rep, cycle_rep, edge_index, cycle_entry_edge,
           cycle_domain, W_ne_lift1, W_ne_lift2, W_ne_lvl1, W_ne_lvl2a,
           W_ne_lvl2b, eps_ne_1, eps_ne_2, W_ec_lift1, W_ec_lift2, W_ec_lvl1,
           W_ec_lvl2a, W_ec_lvl2b, eps_ec_11, eps_ec_12, eps_ec_2, W_mlp):
    N = node_rep.shape[0]
    E = edge_rep.shape[0]
    T = cycle_rep.shape[0]
    C = 20000
    BR = 1000
    T2 = 32 * CH * 47
    E2 = 40 * 8192

    ei = edge_index.astype(jnp.int32)
    CPW = E // (32 * CH)
    i0_3d = ei[0].reshape(32, CPW, CH)
    i1_3d = ei[1].reshape(32, CPW, CH)
    i0_3dt = ei[0].reshape(32 * CPW, 1, CH)
    i1_3dt = ei[1].reshape(32 * CPW, 1, CH)
    cee = cycle_entry_edge.astype(jnp.int32)
    dom = cycle_domain.astype(jnp.int32)
    cee_p = jnp.concatenate([cee, jnp.full((T2 - T,), E, jnp.int32)])
    dom_p = jnp.concatenate([dom, jnp.full((T2 - T,), C, jnp.int32)])
    cee3d = cee_p.reshape(T2 // CH, 1, CH)
    dom3d = dom_p.reshape(T2 // CH, 1, CH)
    crep_p = jnp.concatenate(
        [cycle_rep, jnp.zeros((T2 - T, H), jnp.float32)], axis=0)

    epsmat = jnp.broadcast_to(
        jnp.stack([1.0 + eps_ne_1, 1.0 + eps_ne_2, 1.0 + eps_ec_11,
                   1.0 + eps_ec_12, 1.0 + eps_ec_2,
                   jnp.float32(0), jnp.float32(0), jnp.float32(0)])[:, None],
        (8, H))
    eps12_16 = jnp.broadcast_to((1.0 + eps_ec_12)[None], (16,))

    g0, g1 = _sc_gather2(node_rep, i0_3d, i1_3d, E)

    def fn_s1(xs, fs, ws, eps_ref):
        g0b, g1b, eb = xs
        la = g0b + g1b
        y1e = _mm(jnp.concatenate([la, eb], axis=1), ws[0])
        x2 = eb * eps_ref[1:2, :] + la
        y1o = _mm(x2, ws[1])
        return [y1e, y1o]

    y1e, p1e, y1o, p1o = _tc_fused(
        E, BR, [g0, g1, edge_rep], [], [W_ne_lvl1, W_ne_lift1], epsmat,
        fn_s1, [(H, True), (2 * H, True)], "s1_edge_lin")

    def fn_s2(xs, fs, ws, eps_ref):
        h1 = _nrm(xs[0], fs[0])
        y2o = _mm(_nrm(xs[1], fs[1]), ws[0])
        return [h1, y2o]

    h1, y2o, p2o = _tc_fused(
        E, BR, [y1e, y1o], [p1e, p1o], [W_ne_lift2], epsmat,
        fn_s2, [(H, False), (H, True)], "s2_edge_lin")

    nacc = _sc_scatter_nodes(h1, i0_3dt, i1_3dt, E, N)
    node_out = _tc_node(node_rep, nacc, epsmat, W_ne_lvl2a, W_ne_lvl2b, N)

    g_pad, mg, mc, rcp = _sc_seg_means(dom3d, C, T2, gather_src=edge_rep,
                                       cee3d=cee3d, linear_src=crep_p)
    gm_pad, cycb_pad = _sc_bcast(dom3d, [mg, mc], C, T2)

    def fn_s5(xs, fs, ws, eps_ref):
        gb, gmb, cb, cbb = xs
        y1c = _mm(jnp.concatenate([gb, gmb, cb], axis=1), ws[0])
        ev = eps_ref[4:5, :]
        x2 = jnp.concatenate([cb * ev + gb, cbb * ev + gmb], axis=1)
        y1k = _mm(x2, ws[1])
        return [y1c, y1k]

    y1c, p1c, y1k, p1k = _tc_fused(
        T, BR, [g_pad, gm_pad, cycle_rep, cycb_pad], [],
        [W_ec_lvl1, W_ec_lift1], epsmat,
        fn_s5, [(H, True), (2 * H, True)], "s5_cyc_lin")

    def fn_s6(xs, fs, ws, eps_ref):
        h2 = _nrm(xs[0], fs[0])
        y2k = _mm(_nrm(xs[1], fs[1]), ws[0])
        return [h2, y2k]

    h2, y2k, p2k = _tc_fused(
        T, BR, [y1c, y1k], [p1c, p1k], [W_ec_lift2], epsmat,
        fn_s6, [(H, False), (H, True)], "s6_cyc_lin")

    def fn_norm_only(xs, fs, ws, eps_ref):
        return [_nrm(xs[0], fs[0])]

    cycle_out, = _tc_fused(T, BR, [y2k], [p2k], [], epsmat,
                           fn_norm_only, [(H, False)], "s9_cyc_out")

    h2p = jnp.concatenate([h2, jnp.zeros((T2 - T, H), jnp.float32)], axis=0)
    mh2 = _sc_seg_means(dom3d, C, T2, linear_src=h2p, rcp_in=rcp)
    val = _sc_val(dom3d, h2p, mh2, eps12_16, C, T2)
    lvlc = _sc_scatter_edges(val, cee_p, E2, T2)[:E]

    def fn_s10(xs, fs, ws, eps_ref):
        x = xs[0] * eps_ref[2:3, :] + xs[1]
        return [_mm(x, ws[0])]

    y1f, p1f = _tc_fused(E, BR, [edge_rep, lvlc], [], [W_ec_lvl2a], epsmat,
                         fn_s10, [(2 * H, True)], "s10_edge2")

    def fn_s11(xs, fs, ws, eps_ref):
        return [_mm(_nrm(xs[0], fs[0]), ws[0])]

    y2f, p2f = _tc_fused(E, BR, [y1f], [p1f], [W_ec_lvl2b], epsmat,
                         fn_s11, [(H, True)], "s11_edge2")

    def fn_s12(xs, fs, ws, eps_ref):
        x = jnp.concatenate([_nrm(xs[0], fs[0]), _nrm(xs[1], fs[1])], axis=1)
        return [_mm(x, ws[0])]

    z, pz = _tc_fused(E, BR, [y2o, y2f], [p2o, p2f], [W_mlp], epsmat,
                      fn_s12, [(H, True)], "s12_final_lin")

    edge_out, = _tc_fused(E, BR, [z], [pz], [], epsmat,
                          fn_norm_only, [(H, False)], "s12b_final_norm")

    return (node_out, edge_out, cycle_out)


# speedup vs baseline: 1.2988x; 1.1037x over previous
import functools

import jax
import jax.numpy as jnp
from jax import lax
from jax.experimental import pallas as pl
from jax.experimental.pallas import tpu as pltpu, tpu_sc as plsc

EPS = 1e-05
H = 128
CH = 80

_SC_PARAMS = pltpu.CompilerParams(needs_layout_passes=False)


def _mesh():
    return plsc.VectorSubcoreMesh(core_axis_name="c", subcore_axis_name="s")


def _f32(shape):
    return jax.ShapeDtypeStruct(shape, jnp.float32)


def _sc_gather2(table, i0_3d, i1_3d, E):
    CPW = E // (32 * CH)

    @functools.partial(
        pl.kernel,
        out_type=(_f32((E, H)), _f32((E, H))),
        mesh=_mesh(),
        compiler_params=_SC_PARAMS,
        scratch_types=[
            pltpu.VMEM((CPW, CH), jnp.int32),
            pltpu.VMEM((CPW, CH), jnp.int32),
            pltpu.VMEM((CH, H), jnp.float32),
            pltpu.VMEM((CH, H), jnp.float32),
            pltpu.SemaphoreType.DMA,
            pltpu.SemaphoreType.DMA,
        ],
    )
    def k(tab_h, i0_h, i1_h, g0_h, g1_h, i0v, i1v, b0, b1, s0, s1):
        cid = lax.axis_index("c")
        sid = lax.axis_index("s")
        w = sid * 2 + cid
        r0 = w * CPW
        pltpu.sync_copy(i0_h.at[w], i0v)
        pltpu.sync_copy(i1_h.at[w], i1v)

        def body(ch, _):
            cp0 = pltpu.async_copy(tab_h.at[i0v.at[ch]], b0, s0)
            cp1 = pltpu.async_copy(tab_h.at[i1v.at[ch]], b1, s1)
            cp0.wait()
            cp1.wait()
            base = (r0 + ch) * CH
            pltpu.sync_copy(b0, g0_h.at[pl.ds(base, CH)])
            pltpu.sync_copy(b1, g1_h.at[pl.ds(base, CH)])
            return 0

        lax.fori_loop(0, CPW, body, 0)

    return k(table, i0_3d, i1_3d)


def _sc_scatter_nodes(h1, i0_3d, i1_3d, E, N):
    CPT = E // (16 * CH)
    NH = N // 2
    ACC = 5120
    STR = 312

    @functools.partial(
        pl.kernel,
        out_type=_f32((N, H)),
        mesh=_mesh(),
        compiler_params=_SC_PARAMS,
        scratch_types=[
            pltpu.VMEM((25, 1, CH), jnp.int32),
            pltpu.VMEM((25, 1, CH), jnp.int32),
            pltpu.VMEM((2, CH, H), jnp.float32),
            pltpu.VMEM((2, CH), jnp.int32),
            pltpu.VMEM((64, H), jnp.float32),
            pltpu.VMEM_SHARED((ACC + 8, H), jnp.float32),
            pltpu.SemaphoreType.DMA,
        ],
    )
    def k(h1_h, i0_h, i1_h, out_h, i0v, i1v, hbuf, idxb, zbuf, nacc, sem):
        cid = lax.axis_index("c")
        sid = lax.axis_index("s")
        zv = jnp.zeros((16,), jnp.float32)

        def zrow(r, _):
            for j in range(H // 16):
                zbuf[r, pl.ds(16 * j, 16)] = zv
            return 0

        lax.fori_loop(0, 64, zrow, 0)
        for b in range(5):
            pltpu.sync_copy(zbuf, nacc.at[pl.ds(sid * 320 + b * 64, 64)])

        @pl.when(sid == 0)
        def _():
            pltpu.sync_copy(zbuf.at[pl.ds(0, 8)], nacc.at[pl.ds(ACC, 8)])

        plsc.subcore_barrier()
        nbase = cid * NH

        cp = pltpu.async_copy(h1_h.at[pl.ds(sid * CPT * CH, CH)],
                              hbuf.at[0], sem)

        def body(ch, _):
            g = ch % 25

            @pl.when(g == 0)
            def _():
                pltpu.sync_copy(i0_h.at[pl.ds(sid * CPT + ch, 25)], i0v)
                pltpu.sync_copy(i1_h.at[pl.ds(sid * CPT + ch, 25)], i1v)

            nxt = (ch + 1) % 2
            cur = ch % 2
            pltpu.make_async_copy(h1_h.at[pl.ds(0, CH)], hbuf.at[cur],
                                  sem).wait()

            @pl.when(ch + 1 < CPT)
            def _():
                base2 = (sid * CPT + ch + 1) * CH
                pltpu.async_copy(h1_h.at[pl.ds(base2, CH)], hbuf.at[nxt],
                                 sem)

            for i5 in range(CH // 16):
                sl = pl.ds(16 * i5, 16)
                v0 = i0v[g, 0, sl] - nbase
                v1 = i1v[g, 0, sl] - nbase
                idxb[0, sl] = jnp.where((v0 >= 0) & (v0 < NH), v0, ACC)
                idxb[1, sl] = jnp.where((v1 >= 0) & (v1 < NH), v1, ACC)
            pltpu.sync_copy(hbuf.at[cur], nacc.at[idxb.at[0]], add=True)
            pltpu.sync_copy(hbuf.at[cur], nacc.at[idxb.at[1]], add=True)
            return 0

        lax.fori_loop(0, CPT, body, 0)
        _ = cp
        plsc.subcore_barrier()
        pltpu.sync_copy(nacc.at[pl.ds(sid * STR, STR)],
                        out_h.at[pl.ds(nbase + sid * STR, STR)])

        @pl.when(sid == 15)
        def _():
            pltpu.sync_copy(nacc.at[pl.ds(4992, 8)],
                            out_h.at[pl.ds(nbase + 4992, 8)])

    return k(h1, i0_3d, i1_3d)


def _sc_seg_means(dom3d, C, T2, gather_src=None, cee3d=None, linear_src=None,
                  rcp_in=None):
    NCHK = T2 // CH
    NPT = NCHK // 16
    SEG = C // 2
    SEGH = SEG // 2
    SEGP = 10240
    ACC = 5120
    STR = ACC // 16
    ZB = STR // 4
    do_g = gather_src is not None
    do_l = linear_src is not None
    do_cnt = rcp_in is None

    outs = []
    if do_g:
        outs.append(_f32((T2, H)))
        outs.append(_f32((2 * SEGP, H)))
    if do_l:
        outs.append(_f32((2 * SEGP, H)))
    if do_cnt:
        outs.append(_f32((2 * SEGP, 16)))

    ins = [dom3d]
    if do_g:
        ins += [gather_src, cee3d]
    if do_l:
        ins += [linear_src]
    if not do_cnt:
        ins += [rcp_in]

    @functools.partial(
        pl.kernel,
        out_type=tuple(outs) if len(outs) > 1 else outs[0],
        mesh=_mesh(),
        compiler_params=_SC_PARAMS,
        scratch_types=[
            pltpu.VMEM((NPT, 1, CH), jnp.int32),
            pltpu.VMEM((NPT, 1, CH), jnp.int32),
            pltpu.VMEM((2, CH), jnp.int32),
            pltpu.VMEM((2, CH, H), jnp.float32),
            pltpu.VMEM((CH, H), jnp.float32),
            pltpu.VMEM((ZB, H), jnp.float32),
            pltpu.VMEM((ZB, 16), jnp.float32),
            pltpu.VMEM_SHARED((ACC + 8, H), jnp.float32),
            pltpu.SemaphoreType.DMA,
        ],
    )
    def k(*refs):
        pos = 0
        dom_h = refs[pos]; pos += 1
        if do_g:
            gsrc_h = refs[pos]; pos += 1
            cee_h = refs[pos]; pos += 1
        if do_l:
            lsrc_h = refs[pos]; pos += 1
        if not do_cnt:
            rcp_h = refs[pos]; pos += 1
        if do_g:
            g_h = refs[pos]; pos += 1
            mg_h = refs[pos]; pos += 1
        if do_l:
            ml_h = refs[pos]; pos += 1
        if do_cnt:
            rcp_h = refs[pos]; pos += 1
        lsrc0_h = gsrc_h if do_g else lsrc_h
        (domsl, ceesl, idxb, vbuf, ones128, zbuf, rcpb, sums,
         sem) = refs[pos:pos + 9]

        cid = lax.axis_index("c")
        sid = lax.axis_index("s")
        zv = jnp.zeros((16,), jnp.float32)
        ov = jnp.ones((16,), jnp.float32)

        def initrow(r, _):
            for j in range(H // 16):
                zbuf[r, pl.ds(16 * j, 16)] = zv
            return 0

        lax.fori_loop(0, ZB, initrow, 0)

        def onesrow(r, _):
            for j in range(H // 16):
                ones128[r, pl.ds(16 * j, 16)] = ov
            return 0

        lax.fori_loop(0, CH, onesrow, 0)
        pltpu.sync_copy(dom_h.at[pl.ds(sid * NPT, NPT)], domsl)
        if do_g:
            pltpu.sync_copy(cee_h.at[pl.ds(sid * NPT, NPT)], ceesl)

        def zero_acc():
            for b in range(4):
                pltpu.sync_copy(zbuf, sums.at[pl.ds(sid * STR + b * ZB, ZB)])

            @pl.when(sid == 0)
            def _():
                pltpu.sync_copy(zbuf.at[pl.ds(0, 8)], sums.at[pl.ds(ACC, 8)])

        def build_idx(kk, half):
            dbase = cid * SEG + half * SEGH
            for i5 in range(CH // 16):
                d16 = domsl[kk, 0, pl.ds(16 * i5, 16)]
                dl = d16 - dbase
                ok = (dl >= 0) & (dl < SEGH)
                idxb[0, pl.ds(16 * i5, 16)] = jnp.where(ok, dl, ACC)

        def fetch(kk, buf):
            if phase_is_gather[0]:
                pltpu.async_copy(gsrc_h.at[ceesl.at[kk, 0]], buf, sem)
            else:
                j = sid * NPT + kk
                pltpu.async_copy(lsrc_h.at[pl.ds(j * CH, CH)], buf, sem)

        phase_is_gather = [False]

        def accumulate(phase, half):
            if phase == 2:
                def body2(kk, _):
                    build_idx(kk, half)
                    pltpu.sync_copy(ones128, sums.at[idxb.at[0]], add=True)
                    return 0

                lax.fori_loop(0, NPT, body2, 0)
                return

            phase_is_gather[0] = phase == 0
            fetch(0, vbuf.at[0])

            def body(kk, _):
                cur = kk % 2
                pltpu.make_async_copy(lsrc0_h.at[pl.ds(0, CH)],
                                      vbuf.at[cur], sem).wait()

                @pl.when(kk + 1 < NPT)
                def _():
                    fetch(kk + 1, vbuf.at[(kk + 1) % 2])

                if phase == 0:
                    j = sid * NPT + kk

                    @pl.when((j % 2) == cid)
                    def _():
                        pltpu.sync_copy(vbuf.at[cur],
                                        g_h.at[pl.ds(j * CH, CH)])
                build_idx(kk, half)
                pltpu.sync_copy(vbuf.at[cur], sums.at[idxb.at[0]], add=True)
                return 0

            lax.fori_loop(0, NPT, body, 0)

        def out_row0(half, b):
            return cid * SEGP + half * SEGH + sid * STR + b * ZB

        def finalize_counts(half):
            for b in range(4):
                r0 = sid * STR + b * ZB
                pltpu.sync_copy(sums.at[pl.ds(r0, ZB)], zbuf)

                def frow(r, _):
                    c16 = zbuf[r, pl.ds(0, 16)]
                    rcpb[r, pl.ds(0, 16)] = 1.0 / jnp.maximum(c16, 1.0)
                    return 0

                lax.fori_loop(0, ZB, frow, 0)
                pltpu.sync_copy(rcpb, rcp_h.at[pl.ds(out_row0(half, b), ZB)])
            lax.fori_loop(0, ZB, initrow, 0)

        def finalize(m_h, half):
            for b in range(4):
                r0 = sid * STR + b * ZB
                pltpu.sync_copy(sums.at[pl.ds(r0, ZB)], zbuf)
                pltpu.sync_copy(rcp_h.at[pl.ds(out_row0(half, b), ZB)], rcpb)

                def frow(r, _):
                    rcp = rcpb[r, pl.ds(0, 16)]
                    for j in range(H // 16):
                        zbuf[r, pl.ds(16 * j, 16)] = \
                            zbuf[r, pl.ds(16 * j, 16)] * rcp
                    return 0

                lax.fori_loop(0, ZB, frow, 0)
                pltpu.sync_copy(zbuf, m_h.at[pl.ds(out_row0(half, b), ZB)])
            lax.fori_loop(0, ZB, initrow, 0)

        plan = []
        if do_cnt:
            plan += [(2, None)]
        if do_g:
            plan += [(0, mg_h)]
        if do_l:
            plan += [(1, ml_h)]
        first = True
        for phase, m_h in plan:
            for half in (0, 1):
                if not first:
                    plsc.subcore_barrier()
                zero_acc()
                plsc.subcore_barrier()
                accumulate(phase, half)
                plsc.subcore_barrier()
                if phase == 2:
                    finalize_counts(half)
                else:
                    finalize(m_h, half)
                first = False

    return k(*ins)


def _sc_bcast(dom3d, means_list, C, T2):
    NCHK = T2 // CH
    NPW = NCHK // 32
    SEG, PAD = 10000, 240
    nm = len(means_list)

    @functools.partial(
        pl.kernel,
        out_type=tuple(_f32((T2, H)) for _ in range(nm)) if nm > 1
        else _f32((T2, H)),
        mesh=_mesh(),
        compiler_params=_SC_PARAMS,
        scratch_types=[
            pltpu.VMEM((1, CH), jnp.int32),
            pltpu.VMEM((2, CH), jnp.int32),
            pltpu.VMEM((CH, H), jnp.float32),
            pltpu.SemaphoreType.DMA,
        ],
    )
    def k(*refs):
        dom_h = refs[0]
        m_hs = refs[1:1 + nm]
        o_hs = refs[1 + nm:1 + 2 * nm]
        domv, idxb, vbuf, sem = refs[1 + 2 * nm:]
        cid = lax.axis_index("c")
        sid = lax.axis_index("s")
        w = sid * 2 + cid

        def body(kk, _):
            j = w * NPW + kk
            pltpu.sync_copy(dom_h.at[j], domv)
            for i5 in range(CH // 16):
                d16 = domv[0, pl.ds(16 * i5, 16)]
                idxb[0, pl.ds(16 * i5, 16)] = \
                    d16 + jnp.where(d16 >= SEG, PAD, 0)
            for mi in range(nm):
                pltpu.async_copy(m_hs[mi].at[idxb.at[0]], vbuf, sem).wait()
                pltpu.sync_copy(vbuf, o_hs[mi].at[pl.ds(j * CH, CH)])
            return 0

        lax.fori_loop(0, NPW, body, 0)

    return k(dom3d, *means_list)


def _sc_val(dom3d, h2p, mh2, scale16, C, T2):
    NCHK = T2 // CH
    NPW = NCHK // 32
    SEG, PAD = 10000, 240

    @functools.partial(
        pl.kernel,
        out_type=_f32((T2, H)),
        mesh=_mesh(),
        compiler_params=_SC_PARAMS,
        scratch_types=[
            pltpu.VMEM((1, CH), jnp.int32),
            pltpu.VMEM((2, CH), jnp.int32),
            pltpu.VMEM((CH, H), jnp.float32),
            pltpu.VMEM((CH, H), jnp.float32),
            pltpu.VMEM((16,), jnp.float32),
            pltpu.SemaphoreType.DMA,
        ],
    )
    def k(dom_h, h2_h, m_h, sc_h, out_h, domv, idxb, b1, b2, scv, sem):
        cid = lax.axis_index("c")
        sid = lax.axis_index("s")
        w = sid * 2 + cid
        pltpu.sync_copy(sc_h, scv)
        ev = scv[pl.ds(0, 16)]

        def body(kk, _):
            j = w * NPW + kk
            pltpu.sync_copy(dom_h.at[j], domv)
            pltpu.sync_copy(h2_h.at[pl.ds(j * CH, CH)], b1)
            for i5 in range(CH // 16):
                d16 = domv[0, pl.ds(16 * i5, 16)]
                idxb[0, pl.ds(16 * i5, 16)] = \
                    d16 + jnp.where(d16 >= SEG, PAD, 0)
            pltpu.async_copy(m_h.at[idxb.at[0]], b2, sem).wait()

            def crow(r, _):
                for j8 in range(H // 16):
                    sl = pl.ds(16 * j8, 16)
                    b1[r, sl] = b1[r, sl] * ev + b2[r, sl]
                return 0

            lax.fori_loop(0, CH, crow, 0)
            pltpu.sync_copy(b1, out_h.at[pl.ds(j * CH, CH)])
            return 0

        lax.fori_loop(0, NPW, body, 0)

    return k(dom3d, h2p, mh2, scale16)


def _sc_scatter_edges(val, cee_flat, E2, T2):
    RNG = 8192
    NPSC = E2 // RNG // 2
    TPT = T2 // 16
    NIT = TPT // 16
    ACC = 8320
    DUMMY = RNG + 8
    GCH = 128
    LSZ = TPT + 2 * GCH

    @functools.partial(
        pl.kernel,
        out_type=_f32((E2, H)),
        mesh=_mesh(),
        compiler_params=_SC_PARAMS,
        scratch_types=[
            pltpu.VMEM((TPT,), jnp.int32),
            pltpu.VMEM((LSZ,), jnp.int32),
            pltpu.VMEM((LSZ // GCH + 1, GCH), jnp.int32),
            pltpu.VMEM((GCH, H), jnp.float32),
            pltpu.VMEM((GCH, H), jnp.float32),
            pltpu.VMEM_SHARED((ACC, H), jnp.float32),
            pltpu.SemaphoreType.DMA,
        ],
    )
    def k(val_h, cee_h, out_h, ceebuf, tlist, e2d, vbuf, zbuf, acc, sem):
        cid = lax.axis_index("c")
        sid = lax.axis_index("s")
        zv = jnp.zeros((16,), jnp.float32)
        iota = lax.iota(jnp.int32, 16)

        def zrow(r, _):
            for j in range(H // 16):
                zbuf[r, pl.ds(16 * j, 16)] = zv
            return 0

        lax.fori_loop(0, GCH, zrow, 0)
        tb = sid * TPT
        pltpu.sync_copy(cee_h.at[pl.ds(tb, TPT)], ceebuf)
        for b in range(4):
            pltpu.sync_copy(zbuf, acc.at[pl.ds(sid * 520 + b * GCH, GCH)])
        pltpu.sync_copy(zbuf.at[pl.ds(0, 8)],
                        acc.at[pl.ds(sid * 520 + 512, 8)])
        plsc.subcore_barrier()

        def one_pass(p, _):
            base = (cid * NPSC + p) * RNG

            def scan(i, m):
                ev16 = ceebuf[pl.ds(16 * i, 16)]
                el = ev16 - base
                ok = (el >= 0) & (el < RNG)
                c16 = plsc.all_reduce_population_count(ok)
                plsc.store_compressed(tlist.at[pl.ds(m, 16)],
                                      16 * i + iota, mask=ok)
                return m + c16[0]

            m = lax.fori_loop(0, NIT, scan, jnp.int32(0))
            for g5 in range(GCH // 16):
                tlist[pl.ds(m + 16 * g5, 16)] = jnp.zeros((16,), jnp.int32)
            nch = (m + GCH - 1) // GCH

            def copy2d(ch2, _):
                for i5 in range(GCH // 16):
                    pos = GCH * ch2 + 16 * i5
                    tloc = tlist[pl.ds(pos, 16)]
                    ee = plsc.load_gather(ceebuf, [tloc]) - base
                    valid = (pos + iota) < m
                    e2d[ch2, pl.ds(16 * i5, 16)] = \
                        jnp.where(valid, ee, DUMMY)
                    tlist[pl.ds(pos, 16)] = tloc + tb
                return 0

            lax.fori_loop(0, nch, copy2d, 0)

            def gsc(ch2, _):
                pltpu.async_copy(
                    val_h.at[tlist.at[pl.ds(GCH * ch2, GCH)]],
                    vbuf, sem).wait()
                pltpu.sync_copy(vbuf, acc.at[e2d.at[ch2]], add=True)
                return 0

            lax.fori_loop(0, nch, gsc, 0)
            plsc.subcore_barrier()
            wr = RNG // 16
            pltpu.sync_copy(acc.at[pl.ds(sid * wr, wr)],
                            out_h.at[pl.ds(base + sid * wr, wr)])
            plsc.subcore_barrier()

            def tz(ch2, _):
                pltpu.sync_copy(zbuf, acc.at[e2d.at[ch2]])
                return 0

            lax.fori_loop(0, nch, tz, 0)
            plsc.subcore_barrier()
            return 0

        lax.fori_loop(0, NPSC, one_pass, 0)

    return k(val, cee_flat)


def _tc_fused(R, BR, ins, stats, weights, epsmat, fn, outs_spec, name):
    nb = R // BR
    n_in, n_st, n_w = len(ins), len(stats), len(weights)

    def body(*refs):
        i = pl.program_id(0)
        in_refs = refs[:n_in]
        st_refs = refs[n_in:n_in + n_st]
        w_refs = refs[n_in + n_st:n_in + n_st + n_w]
        eps_ref = refs[n_in + n_st + n_w]
        rest = refs[n_in + n_st + n_w + 1:]
        n_o = len(outs_spec) + sum(1 for _, ws in outs_spec if ws)
        out_refs = rest[:n_o]
        scr_refs = rest[n_o:]

        @pl.when(i == 0)
        def _():
            for st_ref, scr in zip(st_refs, scr_refs):
                s = jnp.sum(st_ref[...], axis=0)
                mu = s[0:1] / R
                var = s[1:2] / R - mu * mu
                rs = lax.rsqrt(var + EPS)
                scr[0:1, :] = mu
                scr[1:2, :] = rs

        finstats = [(scr[0:1, :], scr[1:2, :]) for scr in scr_refs]
        outs = fn([r[...] for r in in_refs], finstats,
                  [r[...] for r in w_refs], eps_ref)
        oi = 0
        for o, (ko, ws) in zip(outs, outs_spec):
            out_refs[oi][...] = o
            oi += 1
            if ws:
                out_refs[oi][0, 0, :] = jnp.sum(o, axis=0)
                out_refs[oi][0, 1, :] = jnp.sum(o * o, axis=0)
                oi += 1

    in_specs = (
        [pl.BlockSpec((BR, a.shape[1]), lambda i: (i, 0)) for a in ins]
        + [pl.BlockSpec(p.shape, lambda i: (0, 0, 0)) for p in stats]
        + [pl.BlockSpec(w.shape, lambda i: (0, 0)) for w in weights]
        + [pl.BlockSpec(epsmat.shape, lambda i: (0, 0))]
    )
    out_shape, out_specs = [], []
    for ko, ws in outs_spec:
        out_shape.append(_f32((R, ko)))
        out_specs.append(pl.BlockSpec((BR, ko), lambda i: (i, 0)))
        if ws:
            out_shape.append(_f32((nb, 2, ko)))
            out_specs.append(pl.BlockSpec((1, 2, ko), lambda i: (i, 0, 0)))
    scratch = [pltpu.VMEM((2, p.shape[2]), jnp.float32) for p in stats]
    return pl.pallas_call(
        body,
        grid=(nb,),
        in_specs=in_specs,
        out_specs=out_specs,
        out_shape=out_shape,
        scratch_shapes=scratch,
        name=name,
    )(*ins, *stats, *weights, epsmat)


def _nrm(y, st):
    mu, rs = st
    return jnp.maximum((y - mu) * rs, 0.0)


def _mm(x, w):
    return lax.dot_general(x, w, (((1,), (1,)), ((), ())),
                           preferred_element_type=jnp.float32)


def _tc_node(node_rep, partials, epsmat, Wa, Wb, N):
    def body(x_ref, p_ref, eps_ref, wa_ref, wb_ref, o_ref):
        ev = eps_ref[0:1, :]
        x = x_ref[...] * ev + p_ref[...]
        y1 = _mm(x, wa_ref[...])
        mu = jnp.mean(y1, axis=0, keepdims=True)
        var = jnp.mean((y1 - mu) ** 2, axis=0, keepdims=True)
        h = jnp.maximum((y1 - mu) * lax.rsqrt(var + EPS), 0.0)
        y2 = _mm(h, wb_ref[...])
        mu2 = jnp.mean(y2, axis=0, keepdims=True)
        var2 = jnp.mean((y2 - mu2) ** 2, axis=0, keepdims=True)
        o_ref[...] = jnp.maximum((y2 - mu2) * lax.rsqrt(var2 + EPS), 0.0)

    return pl.pallas_call(
        body,
        out_shape=_f32((N, H)),
        name="node_mlp2",
    )(node_rep, partials, epsmat, Wa, Wb)


def kernel(node_rep, edge_rep, cycle_rep, edge_index, cycle_entry_edge,
           cycle_domain, W_ne_lift1, W_ne_lift2, W_ne_lvl1, W_ne_lvl2a,
           W_ne_lvl2b, eps_ne_1, eps_ne_2, W_ec_lift1, W_ec_lift2, W_ec_lvl1,
           W_ec_lvl2a, W_ec_lvl2b, eps_ec_11, eps_ec_12, eps_ec_2, W_mlp):
    N = node_rep.shape[0]
    E = edge_rep.shape[0]
    T = cycle_rep.shape[0]
    C = 20000
    BR = 1000
    T2 = 32 * CH * 47
    E2 = 40 * 8192

    ei = edge_index.astype(jnp.int32)
    CPW = E // (32 * CH)
    i0_3d = ei[0].reshape(32, CPW, CH)
    i1_3d = ei[1].reshape(32, CPW, CH)
    i0_3dt = ei[0].reshape(32 * CPW, 1, CH)
    i1_3dt = ei[1].reshape(32 * CPW, 1, CH)
    cee = cycle_entry_edge.astype(jnp.int32)
    dom = cycle_domain.astype(jnp.int32)
    cee_p = jnp.concatenate([cee, jnp.full((T2 - T,), E, jnp.int32)])
    dom_p = jnp.concatenate([dom, jnp.full((T2 - T,), C, jnp.int32)])
    cee3d = cee_p.reshape(T2 // CH, 1, CH)
    dom3d = dom_p.reshape(T2 // CH, 1, CH)
    crep_p = jnp.concatenate(
        [cycle_rep, jnp.zeros((T2 - T, H), jnp.float32)], axis=0)

    epsmat = jnp.broadcast_to(
        jnp.stack([1.0 + eps_ne_1, 1.0 + eps_ne_2, 1.0 + eps_ec_11,
                   1.0 + eps_ec_12, 1.0 + eps_ec_2,
                   jnp.float32(0), jnp.float32(0), jnp.float32(0)])[:, None],
        (8, H))
    eps12_16 = jnp.broadcast_to((1.0 + eps_ec_12)[None], (16,))

    g0, g1 = _sc_gather2(node_rep, i0_3d, i1_3d, E)

    def fn_s1(xs, fs, ws, eps_ref):
        g0b, g1b, eb = xs
        la = g0b + g1b
        y1e = _mm(jnp.concatenate([la, eb], axis=1), ws[0])
        x2 = eb * eps_ref[1:2, :] + la
        y1o = _mm(x2, ws[1])
        return [y1e, y1o]

    y1e, p1e, y1o, p1o = _tc_fused(
        E, BR, [g0, g1, edge_rep], [], [W_ne_lvl1, W_ne_lift1], epsmat,
        fn_s1, [(H, True), (2 * H, True)], "s1_edge_lin")

    def fn_s2(xs, fs, ws, eps_ref):
        h1 = _nrm(xs[0], fs[0])
        y2o = _mm(_nrm(xs[1], fs[1]), ws[0])
        return [h1, y2o]

    h1, y2o, p2o = _tc_fused(
        E, BR, [y1e, y1o], [p1e, p1o], [W_ne_lift2], epsmat,
        fn_s2, [(H, False), (H, True)], "s2_edge_lin")

    nacc = _sc_scatter_nodes(h1, i0_3dt, i1_3dt, E, N)
    node_out = _tc_node(node_rep, nacc, epsmat, W_ne_lvl2a, W_ne_lvl2b, N)

    g_pad, mg, mc, rcp = _sc_seg_means(dom3d, C, T2, gather_src=edge_rep,
                                       cee3d=cee3d, linear_src=crep_p)
    gm_pad, cycb_pad = _sc_bcast(dom3d, [mg, mc], C, T2)

    def fn_s5(xs, fs, ws, eps_ref):
        gb, gmb, cb, cbb = xs
        y1c = _mm(jnp.concatenate([gb, gmb, cb], axis=1), ws[0])
        ev = eps_ref[4:5, :]
        x2 = jnp.concatenate([cb * ev + gb, cbb * ev + gmb], axis=1)
        y1k = _mm(x2, ws[1])
        return [y1c, y1k]

    y1c, p1c, y1k, p1k = _tc_fused(
        T, BR, [g_pad, gm_pad, cycle_rep, cycb_pad], [],
        [W_ec_lvl1, W_ec_lift1], epsmat,
        fn_s5, [(H, True), (2 * H, True)], "s5_cyc_lin")

    def fn_s6(xs, fs, ws, eps_ref):
        h2 = _nrm(xs[0], fs[0])
        y2k = _mm(_nrm(xs[1], fs[1]), ws[0])
        return [h2, y2k]

    h2, y2k, p2k = _tc_fused(
        T, BR, [y1c, y1k], [p1c, p1k], [W_ec_lift2], epsmat,
        fn_s6, [(H, False), (H, True)], "s6_cyc_lin")

    def fn_norm_only(xs, fs, ws, eps_ref):
        return [_nrm(xs[0], fs[0])]

    cycle_out, = _tc_fused(T, BR, [y2k], [p2k], [], epsmat,
                           fn_norm_only, [(H, False)], "s9_cyc_out")

    h2p = jnp.concatenate([h2, jnp.zeros((T2 - T, H), jnp.float32)], axis=0)
    mh2 = _sc_seg_means(dom3d, C, T2, linear_src=h2p, rcp_in=rcp)
    val = _sc_val(dom3d, h2p, mh2, eps12_16, C, T2)
    lvlc = _sc_scatter_edges(val, cee_p, E2, T2)[:E]

    def fn_s10(xs, fs, ws, eps_ref):
        x = xs[0] * eps_ref[2:3, :] + xs[1]
        return [_mm(x, ws[0])]

    y1f, p1f = _tc_fused(E, BR, [edge_rep, lvlc], [], [W_ec_lvl2a], epsmat,
                         fn_s10, [(2 * H, True)], "s10_edge2")

    def fn_s11(xs, fs, ws, eps_ref):
        return [_mm(_nrm(xs[0], fs[0]), ws[0])]

    y2f, p2f = _tc_fused(E, BR, [y1f], [p1f], [W_ec_lvl2b], epsmat,
                         fn_s11, [(H, True)], "s11_edge2")

    def fn_s12(xs, fs, ws, eps_ref):
        x = jnp.concatenate([_nrm(xs[0], fs[0]), _nrm(xs[1], fs[1])], axis=1)
        return [_mm(x, ws[0])]

    z, pz = _tc_fused(E, BR, [y2o, y2f], [p2o, p2f], [W_mlp], epsmat,
                      fn_s12, [(H, True)], "s12_final_lin")

    edge_out, = _tc_fused(E, BR, [z], [pz], [], epsmat,
                          fn_norm_only, [(H, False)], "s12b_final_norm")

    return (node_out, edge_out, cycle_out)


# speedup vs baseline: 1.3405x; 1.0321x over previous
import functools

import jax
import jax.numpy as jnp
from jax import lax
from jax.experimental import pallas as pl
from jax.experimental.pallas import tpu as pltpu, tpu_sc as plsc

EPS = 1e-05
H = 128
CH = 80

_SC_PARAMS = pltpu.CompilerParams(needs_layout_passes=False)


def _mesh():
    return plsc.VectorSubcoreMesh(core_axis_name="c", subcore_axis_name="s")


def _f32(shape):
    return jax.ShapeDtypeStruct(shape, jnp.float32)


def _sc_gather2(table, i0_3d, i1_3d, E):
    CPW = E // (32 * CH)

    @functools.partial(
        pl.kernel,
        out_type=(_f32((E, H)), _f32((E, H))),
        mesh=_mesh(),
        compiler_params=_SC_PARAMS,
        scratch_types=[
            pltpu.VMEM((CPW, CH), jnp.int32),
            pltpu.VMEM((CPW, CH), jnp.int32),
            pltpu.VMEM((2, CH, H), jnp.float32),
            pltpu.VMEM((2, CH, H), jnp.float32),
            pltpu.SemaphoreType.DMA,
            pltpu.SemaphoreType.DMA,
        ],
    )
    def k(tab_h, i0_h, i1_h, g0_h, g1_h, i0v, i1v, b0, b1, s0, s1):
        cid = lax.axis_index("c")
        sid = lax.axis_index("s")
        w = sid * 2 + cid
        r0 = w * CPW
        pltpu.sync_copy(i0_h.at[w], i0v)
        pltpu.sync_copy(i1_h.at[w], i1v)
        pltpu.async_copy(tab_h.at[i0v.at[0]], b0.at[0], s0)
        pltpu.async_copy(tab_h.at[i1v.at[0]], b1.at[0], s1)

        def body(ch, _):
            cur = ch % 2
            pltpu.make_async_copy(g0_h.at[pl.ds(0, CH)], b0.at[cur],
                                  s0).wait()
            pltpu.make_async_copy(g0_h.at[pl.ds(0, CH)], b1.at[cur],
                                  s1).wait()

            @pl.when(ch + 1 < CPW)
            def _():
                nxt = (ch + 1) % 2
                pltpu.async_copy(tab_h.at[i0v.at[ch + 1]], b0.at[nxt], s0)
                pltpu.async_copy(tab_h.at[i1v.at[ch + 1]], b1.at[nxt], s1)

            base = (r0 + ch) * CH
            pltpu.sync_copy(b0.at[cur], g0_h.at[pl.ds(base, CH)])
            pltpu.sync_copy(b1.at[cur], g1_h.at[pl.ds(base, CH)])
            return 0

        lax.fori_loop(0, CPW, body, 0)

    return k(table, i0_3d, i1_3d)


def _sc_scatter_nodes(h1, i0_3d, i1_3d, E, N):
    CPT = E // (16 * CH)
    NH = N // 2
    ACC = 5120
    STR = 312

    @functools.partial(
        pl.kernel,
        out_type=_f32((N, H)),
        mesh=_mesh(),
        compiler_params=_SC_PARAMS,
        scratch_types=[
            pltpu.VMEM((25, 1, CH), jnp.int32),
            pltpu.VMEM((25, 1, CH), jnp.int32),
            pltpu.VMEM((2, CH, H), jnp.float32),
            pltpu.VMEM((2, CH), jnp.int32),
            pltpu.VMEM((64, H), jnp.float32),
            pltpu.VMEM_SHARED((ACC + 8, H), jnp.float32),
            pltpu.SemaphoreType.DMA,
        ],
    )
    def k(h1_h, i0_h, i1_h, out_h, i0v, i1v, hbuf, idxb, zbuf, nacc, sem):
        cid = lax.axis_index("c")
        sid = lax.axis_index("s")
        zv = jnp.zeros((16,), jnp.float32)

        def zrow(r, _):
            for j in range(H // 16):
                zbuf[r, pl.ds(16 * j, 16)] = zv
            return 0

        lax.fori_loop(0, 64, zrow, 0)
        for b in range(5):
            pltpu.sync_copy(zbuf, nacc.at[pl.ds(sid * 320 + b * 64, 64)])

        @pl.when(sid == 0)
        def _():
            pltpu.sync_copy(zbuf.at[pl.ds(0, 8)], nacc.at[pl.ds(ACC, 8)])

        plsc.subcore_barrier()
        nbase = cid * NH

        cp = pltpu.async_copy(h1_h.at[pl.ds(sid * CPT * CH, CH)],
                              hbuf.at[0], sem)

        def body(ch, _):
            g = ch % 25

            @pl.when(g == 0)
            def _():
                pltpu.sync_copy(i0_h.at[pl.ds(sid * CPT + ch, 25)], i0v)
                pltpu.sync_copy(i1_h.at[pl.ds(sid * CPT + ch, 25)], i1v)

            nxt = (ch + 1) % 2
            cur = ch % 2
            pltpu.make_async_copy(h1_h.at[pl.ds(0, CH)], hbuf.at[cur],
                                  sem).wait()

            @pl.when(ch + 1 < CPT)
            def _():
                base2 = (sid * CPT + ch + 1) * CH
                pltpu.async_copy(h1_h.at[pl.ds(base2, CH)], hbuf.at[nxt],
                                 sem)

            for i5 in range(CH // 16):
                sl = pl.ds(16 * i5, 16)
                v0 = i0v[g, 0, sl] - nbase
                v1 = i1v[g, 0, sl] - nbase
                idxb[0, sl] = jnp.where((v0 >= 0) & (v0 < NH), v0, ACC)
                idxb[1, sl] = jnp.where((v1 >= 0) & (v1 < NH), v1, ACC)
            pltpu.sync_copy(hbuf.at[cur], nacc.at[idxb.at[0]], add=True)
            pltpu.sync_copy(hbuf.at[cur], nacc.at[idxb.at[1]], add=True)
            return 0

        lax.fori_loop(0, CPT, body, 0)
        _ = cp
        plsc.subcore_barrier()
        pltpu.sync_copy(nacc.at[pl.ds(sid * STR, STR)],
                        out_h.at[pl.ds(nbase + sid * STR, STR)])

        @pl.when(sid == 15)
        def _():
            pltpu.sync_copy(nacc.at[pl.ds(4992, 8)],
                            out_h.at[pl.ds(nbase + 4992, 8)])

    return k(h1, i0_3d, i1_3d)


def _sc_seg_means(dom3d, C, T2, gather_src=None, cee3d=None, linear_src=None,
                  rcp_in=None):
    NCHK = T2 // CH
    NPT = NCHK // 16
    SEG = C // 2
    SEGH = SEG // 2
    SEGP = 10240
    ACC = 5120
    STR = ACC // 16
    ZB = STR // 4
    do_g = gather_src is not None
    do_l = linear_src is not None
    do_cnt = rcp_in is None

    outs = []
    if do_g:
        outs.append(_f32((T2, H)))
        outs.append(_f32((2 * SEGP, H)))
    if do_l:
        outs.append(_f32((2 * SEGP, H)))
    if do_cnt:
        outs.append(_f32((2 * SEGP, 16)))

    ins = [dom3d]
    if do_g:
        ins += [gather_src, cee3d]
    if do_l:
        ins += [linear_src]
    if not do_cnt:
        ins += [rcp_in]

    @functools.partial(
        pl.kernel,
        out_type=tuple(outs) if len(outs) > 1 else outs[0],
        mesh=_mesh(),
        compiler_params=_SC_PARAMS,
        scratch_types=[
            pltpu.VMEM((NPT, 1, CH), jnp.int32),
            pltpu.VMEM((NPT, 1, CH), jnp.int32),
            pltpu.VMEM((2, CH), jnp.int32),
            pltpu.VMEM((2, CH, H), jnp.float32),
            pltpu.VMEM((CH, H), jnp.float32),
            pltpu.VMEM((ZB, H), jnp.float32),
            pltpu.VMEM((ZB, 16), jnp.float32),
            pltpu.VMEM_SHARED((ACC + 8, H), jnp.float32),
            pltpu.SemaphoreType.DMA,
        ],
    )
    def k(*refs):
        pos = 0
        dom_h = refs[pos]; pos += 1
        if do_g:
            gsrc_h = refs[pos]; pos += 1
            cee_h = refs[pos]; pos += 1
        if do_l:
            lsrc_h = refs[pos]; pos += 1
        if not do_cnt:
            rcp_h = refs[pos]; pos += 1
        if do_g:
            g_h = refs[pos]; pos += 1
            mg_h = refs[pos]; pos += 1
        if do_l:
            ml_h = refs[pos]; pos += 1
        if do_cnt:
            rcp_h = refs[pos]; pos += 1
        lsrc0_h = gsrc_h if do_g else lsrc_h
        (domsl, ceesl, idxb, vbuf, ones128, zbuf, rcpb, sums,
         sem) = refs[pos:pos + 9]

        cid = lax.axis_index("c")
        sid = lax.axis_index("s")
        zv = jnp.zeros((16,), jnp.float32)
        ov = jnp.ones((16,), jnp.float32)

        def initrow(r, _):
            for j in range(H // 16):
                zbuf[r, pl.ds(16 * j, 16)] = zv
            return 0

        lax.fori_loop(0, ZB, initrow, 0)

        def onesrow(r, _):
            for j in range(H // 16):
                ones128[r, pl.ds(16 * j, 16)] = ov
            return 0

        lax.fori_loop(0, CH, onesrow, 0)
        pltpu.sync_copy(dom_h.at[pl.ds(sid * NPT, NPT)], domsl)
        if do_g:
            pltpu.sync_copy(cee_h.at[pl.ds(sid * NPT, NPT)], ceesl)

        def zero_acc():
            for b in range(4):
                pltpu.sync_copy(zbuf, sums.at[pl.ds(sid * STR + b * ZB, ZB)])

            @pl.when(sid == 0)
            def _():
                pltpu.sync_copy(zbuf.at[pl.ds(0, 8)], sums.at[pl.ds(ACC, 8)])

        def build_idx(kk, half):
            dbase = cid * SEG + half * SEGH
            for i5 in range(CH // 16):
                d16 = domsl[kk, 0, pl.ds(16 * i5, 16)]
                dl = d16 - dbase
                ok = (dl >= 0) & (dl < SEGH)
                idxb[0, pl.ds(16 * i5, 16)] = jnp.where(ok, dl, ACC)

        def fetch(kk, buf):
            if phase_is_gather[0]:
                pltpu.async_copy(gsrc_h.at[ceesl.at[kk, 0]], buf, sem)
            else:
                j = sid * NPT + kk
                pltpu.async_copy(lsrc_h.at[pl.ds(j * CH, CH)], buf, sem)

        phase_is_gather = [False]

        def accumulate(phase, half):
            if phase == 2:
                def body2(kk, _):
                    build_idx(kk, half)
                    pltpu.sync_copy(ones128, sums.at[idxb.at[0]], add=True)
                    return 0

                lax.fori_loop(0, NPT, body2, 0)
                return

            phase_is_gather[0] = phase == 0
            fetch(0, vbuf.at[0])

            def body(kk, _):
                cur = kk % 2
                pltpu.make_async_copy(lsrc0_h.at[pl.ds(0, CH)],
                                      vbuf.at[cur], sem).wait()

                @pl.when(kk + 1 < NPT)
                def _():
                    fetch(kk + 1, vbuf.at[(kk + 1) % 2])

                if phase == 0:
                    j = sid * NPT + kk

                    @pl.when((j % 2) == cid)
                    def _():
                        pltpu.sync_copy(vbuf.at[cur],
                                        g_h.at[pl.ds(j * CH, CH)])
                build_idx(kk, half)
                pltpu.sync_copy(vbuf.at[cur], sums.at[idxb.at[0]], add=True)
                return 0

            lax.fori_loop(0, NPT, body, 0)

        def out_row0(half, b):
            return cid * SEGP + half * SEGH + sid * STR + b * ZB

        def finalize_counts(half):
            for b in range(4):
                r0 = sid * STR + b * ZB
                pltpu.sync_copy(sums.at[pl.ds(r0, ZB)], zbuf)

                def frow(r, _):
                    c16 = zbuf[r, pl.ds(0, 16)]
                    rcpb[r, pl.ds(0, 16)] = 1.0 / jnp.maximum(c16, 1.0)
                    return 0

                lax.fori_loop(0, ZB, frow, 0)
                pltpu.sync_copy(rcpb, rcp_h.at[pl.ds(out_row0(half, b), ZB)])
            lax.fori_loop(0, ZB, initrow, 0)

        def finalize(m_h, half):
            for b in range(4):
                r0 = sid * STR + b * ZB
                pltpu.sync_copy(sums.at[pl.ds(r0, ZB)], zbuf)
                pltpu.sync_copy(rcp_h.at[pl.ds(out_row0(half, b), ZB)], rcpb)

                def frow(r, _):
                    rcp = rcpb[r, pl.ds(0, 16)]
                    for j in range(H // 16):
                        zbuf[r, pl.ds(16 * j, 16)] = \
                            zbuf[r, pl.ds(16 * j, 16)] * rcp
                    return 0

                lax.fori_loop(0, ZB, frow, 0)
                pltpu.sync_copy(zbuf, m_h.at[pl.ds(out_row0(half, b), ZB)])
            lax.fori_loop(0, ZB, initrow, 0)

        plan = []
        if do_cnt:
            plan += [(2, None)]
        if do_g:
            plan += [(0, mg_h)]
        if do_l:
            plan += [(1, ml_h)]
        first = True
        for phase, m_h in plan:
            for half in (0, 1):
                if not first:
                    plsc.subcore_barrier()
                zero_acc()
                plsc.subcore_barrier()
                accumulate(phase, half)
                plsc.subcore_barrier()
                if phase == 2:
                    finalize_counts(half)
                else:
                    finalize(m_h, half)
                first = False

    return k(*ins)


def _sc_bcast(dom3d, means_list, C, T2):
    NCHK = T2 // CH
    NPW = NCHK // 32
    SEG, PAD = 10000, 240
    nm = len(means_list)

    @functools.partial(
        pl.kernel,
        out_type=tuple(_f32((T2, H)) for _ in range(nm)) if nm > 1
        else _f32((T2, H)),
        mesh=_mesh(),
        compiler_params=_SC_PARAMS,
        scratch_types=[
            pltpu.VMEM((NPW, 1, CH), jnp.int32),
            pltpu.VMEM((2, CH), jnp.int32),
            pltpu.VMEM((2, 2, CH, H), jnp.float32),
            pltpu.SemaphoreType.DMA,
        ],
    )
    def k(*refs):
        dom_h = refs[0]
        m_hs = refs[1:1 + nm]
        o_hs = refs[1 + nm:1 + 2 * nm]
        domsl, idxb, vbuf, sem = refs[1 + 2 * nm:]
        cid = lax.axis_index("c")
        sid = lax.axis_index("s")
        w = sid * 2 + cid
        pltpu.sync_copy(dom_h.at[pl.ds(w * NPW, NPW)], domsl)

        def build(kk, slot):
            for i5 in range(CH // 16):
                d16 = domsl[kk, 0, pl.ds(16 * i5, 16)]
                idxb[slot, pl.ds(16 * i5, 16)] = \
                    d16 + jnp.where(d16 >= SEG, PAD, 0)

        def fire(kk, slot):
            for mi in range(nm):
                pltpu.async_copy(m_hs[mi].at[idxb.at[slot]],
                                 vbuf.at[slot, mi], sem)

        build(0, 0)
        fire(0, 0)

        def body(kk, _):
            cur = kk % 2
            for mi in range(nm):
                pltpu.make_async_copy(o_hs[0].at[pl.ds(0, CH)],
                                      vbuf.at[cur, mi], sem).wait()

            @pl.when(kk + 1 < NPW)
            def _():
                build(kk + 1, (kk + 1) % 2)
                fire(kk + 1, (kk + 1) % 2)

            j = w * NPW + kk
            for mi in range(nm):
                pltpu.sync_copy(vbuf.at[cur, mi],
                                o_hs[mi].at[pl.ds(j * CH, CH)])
            return 0

        lax.fori_loop(0, NPW, body, 0)

    return k(dom3d, *means_list)


def _sc_val(dom3d, h2p, mh2, scale16, C, T2):
    NCHK = T2 // CH
    NPW = NCHK // 32
    SEG, PAD = 10000, 240

    @functools.partial(
        pl.kernel,
        out_type=_f32((T2, H)),
        mesh=_mesh(),
        compiler_params=_SC_PARAMS,
        scratch_types=[
            pltpu.VMEM((1, CH), jnp.int32),
            pltpu.VMEM((2, CH), jnp.int32),
            pltpu.VMEM((CH, H), jnp.float32),
            pltpu.VMEM((CH, H), jnp.float32),
            pltpu.VMEM((16,), jnp.float32),
            pltpu.SemaphoreType.DMA,
        ],
    )
    def k(dom_h, h2_h, m_h, sc_h, out_h, domv, idxb, b1, b2, scv, sem):
        cid = lax.axis_index("c")
        sid = lax.axis_index("s")
        w = sid * 2 + cid
        pltpu.sync_copy(sc_h, scv)
        ev = scv[pl.ds(0, 16)]

        def body(kk, _):
            j = w * NPW + kk
            pltpu.sync_copy(dom_h.at[j], domv)
            pltpu.sync_copy(h2_h.at[pl.ds(j * CH, CH)], b1)
            for i5 in range(CH // 16):
                d16 = domv[0, pl.ds(16 * i5, 16)]
                idxb[0, pl.ds(16 * i5, 16)] = \
                    d16 + jnp.where(d16 >= SEG, PAD, 0)
            pltpu.async_copy(m_h.at[idxb.at[0]], b2, sem).wait()

            def crow(r, _):
                for j8 in range(H // 16):
                    sl = pl.ds(16 * j8, 16)
                    b1[r, sl] = b1[r, sl] * ev + b2[r, sl]
                return 0

            lax.fori_loop(0, CH, crow, 0)
            pltpu.sync_copy(b1, out_h.at[pl.ds(j * CH, CH)])
            return 0

        lax.fori_loop(0, NPW, body, 0)

    return k(dom3d, h2p, mh2, scale16)


def _sc_scatter_edges(val, cee_flat, E2, T2):
    RNG = 8192
    NPSC = E2 // RNG // 2
    TPT = T2 // 16
    NIT = TPT // 16
    ACC = 8320
    DUMMY = RNG + 8
    GCH = 128
    LSZ = TPT + 2 * GCH

    @functools.partial(
        pl.kernel,
        out_type=_f32((E2, H)),
        mesh=_mesh(),
        compiler_params=_SC_PARAMS,
        scratch_types=[
            pltpu.VMEM((TPT,), jnp.int32),
            pltpu.VMEM((LSZ,), jnp.int32),
            pltpu.VMEM((LSZ // GCH + 1, GCH), jnp.int32),
            pltpu.VMEM((GCH, H), jnp.float32),
            pltpu.VMEM((GCH, H), jnp.float32),
            pltpu.VMEM_SHARED((ACC, H), jnp.float32),
            pltpu.SemaphoreType.DMA,
        ],
    )
    def k(val_h, cee_h, out_h, ceebuf, tlist, e2d, vbuf, zbuf, acc, sem):
        cid = lax.axis_index("c")
        sid = lax.axis_index("s")
        zv = jnp.zeros((16,), jnp.float32)
        iota = lax.iota(jnp.int32, 16)

        def zrow(r, _):
            for j in range(H // 16):
                zbuf[r, pl.ds(16 * j, 16)] = zv
            return 0

        lax.fori_loop(0, GCH, zrow, 0)
        tb = sid * TPT
        pltpu.sync_copy(cee_h.at[pl.ds(tb, TPT)], ceebuf)
        for b in range(4):
            pltpu.sync_copy(zbuf, acc.at[pl.ds(sid * 520 + b * GCH, GCH)])
        pltpu.sync_copy(zbuf.at[pl.ds(0, 8)],
                        acc.at[pl.ds(sid * 520 + 512, 8)])
        plsc.subcore_barrier()

        def one_pass(p, _):
            base = (cid * NPSC + p) * RNG

            def scan(i, m):
                ev16 = ceebuf[pl.ds(16 * i, 16)]
                el = ev16 - base
                ok = (el >= 0) & (el < RNG)
                c16 = plsc.all_reduce_population_count(ok)
                plsc.store_compressed(tlist.at[pl.ds(m, 16)],
                                      16 * i + iota, mask=ok)
                return m + c16[0]

            m = lax.fori_loop(0, NIT, scan, jnp.int32(0))
            for g5 in range(GCH // 16):
                tlist[pl.ds(m + 16 * g5, 16)] = jnp.zeros((16,), jnp.int32)
            nch = (m + GCH - 1) // GCH

            def copy2d(ch2, _):
                for i5 in range(GCH // 16):
                    pos = GCH * ch2 + 16 * i5
                    tloc = tlist[pl.ds(pos, 16)]
                    ee = plsc.load_gather(ceebuf, [tloc]) - base
                    valid = (pos + iota) < m
                    e2d[ch2, pl.ds(16 * i5, 16)] = \
                        jnp.where(valid, ee, DUMMY)
                    tlist[pl.ds(pos, 16)] = tloc + tb
                return 0

            lax.fori_loop(0, nch, copy2d, 0)

            def gsc(ch2, _):
                pltpu.async_copy(
                    val_h.at[tlist.at[pl.ds(GCH * ch2, GCH)]],
                    vbuf, sem).wait()
                pltpu.sync_copy(vbuf, acc.at[e2d.at[ch2]], add=True)
                return 0

            lax.fori_loop(0, nch, gsc, 0)
            plsc.subcore_barrier()
            wr = RNG // 16
            pltpu.sync_copy(acc.at[pl.ds(sid * wr, wr)],
                            out_h.at[pl.ds(base + sid * wr, wr)])
            plsc.subcore_barrier()

            def tz(ch2, _):
                pltpu.sync_copy(zbuf, acc.at[e2d.at[ch2]])
                return 0

            lax.fori_loop(0, nch, tz, 0)
            plsc.subcore_barrier()
            return 0

        lax.fori_loop(0, NPSC, one_pass, 0)

    return k(val, cee_flat)


def _tc_fused(R, BR, ins, stats, weights, epsmat, fn, outs_spec, name):
    nb = R // BR
    n_in, n_st, n_w = len(ins), len(stats), len(weights)

    def body(*refs):
        i = pl.program_id(0)
        in_refs = refs[:n_in]
        st_refs = refs[n_in:n_in + n_st]
        w_refs = refs[n_in + n_st:n_in + n_st + n_w]
        eps_ref = refs[n_in + n_st + n_w]
        rest = refs[n_in + n_st + n_w + 1:]
        n_o = len(outs_spec) + sum(1 for _, ws in outs_spec if ws)
        out_refs = rest[:n_o]
        scr_refs = rest[n_o:]

        @pl.when(i == 0)
        def _():
            for st_ref, scr in zip(st_refs, scr_refs):
                s = jnp.sum(st_ref[...], axis=0)
                mu = s[0:1] / R
                var = s[1:2] / R - mu * mu
                rs = lax.rsqrt(var + EPS)
                scr[0:1, :] = mu
                scr[1:2, :] = rs

        finstats = [(scr[0:1, :], scr[1:2, :]) for scr in scr_refs]
        outs = fn([r[...] for r in in_refs], finstats,
                  [r[...] for r in w_refs], eps_ref)
        oi = 0
        for o, (ko, ws) in zip(outs, outs_spec):
            out_refs[oi][...] = o
            oi += 1
            if ws:
                out_refs[oi][0, 0, :] = jnp.sum(o, axis=0)
                out_refs[oi][0, 1, :] = jnp.sum(o * o, axis=0)
                oi += 1

    in_specs = (
        [pl.BlockSpec((BR, a.shape[1]), lambda i: (i, 0)) for a in ins]
        + [pl.BlockSpec(p.shape, lambda i: (0, 0, 0)) for p in stats]
        + [pl.BlockSpec(w.shape, lambda i: (0, 0)) for w in weights]
        + [pl.BlockSpec(epsmat.shape, lambda i: (0, 0))]
    )
    out_shape, out_specs = [], []
    for ko, ws in outs_spec:
        out_shape.append(_f32((R, ko)))
        out_specs.append(pl.BlockSpec((BR, ko), lambda i: (i, 0)))
        if ws:
            out_shape.append(_f32((nb, 2, ko)))
            out_specs.append(pl.BlockSpec((1, 2, ko), lambda i: (i, 0, 0)))
    scratch = [pltpu.VMEM((2, p.shape[2]), jnp.float32) for p in stats]
    return pl.pallas_call(
        body,
        grid=(nb,),
        in_specs=in_specs,
        out_specs=out_specs,
        out_shape=out_shape,
        scratch_shapes=scratch,
        name=name,
    )(*ins, *stats, *weights, epsmat)


def _nrm(y, st):
    mu, rs = st
    return jnp.maximum((y - mu) * rs, 0.0)


def _mm(x, w):
    return lax.dot_general(x, w, (((1,), (1,)), ((), ())),
                           preferred_element_type=jnp.float32)


def _tc_node(node_rep, partials, epsmat, Wa, Wb, N):
    def body(x_ref, p_ref, eps_ref, wa_ref, wb_ref, o_ref):
        ev = eps_ref[0:1, :]
        x = x_ref[...] * ev + p_ref[...]
        y1 = _mm(x, wa_ref[...])
        mu = jnp.mean(y1, axis=0, keepdims=True)
        var = jnp.mean((y1 - mu) ** 2, axis=0, keepdims=True)
        h = jnp.maximum((y1 - mu) * lax.rsqrt(var + EPS), 0.0)
        y2 = _mm(h, wb_ref[...])
        mu2 = jnp.mean(y2, axis=0, keepdims=True)
        var2 = jnp.mean((y2 - mu2) ** 2, axis=0, keepdims=True)
        o_ref[...] = jnp.maximum((y2 - mu2) * lax.rsqrt(var2 + EPS), 0.0)

    return pl.pallas_call(
        body,
        out_shape=_f32((N, H)),
        name="node_mlp2",
    )(node_rep, partials, epsmat, Wa, Wb)


def kernel(node_rep, edge_rep, cycle_rep, edge_index, cycle_entry_edge,
           cycle_domain, W_ne_lift1, W_ne_lift2, W_ne_lvl1, W_ne_lvl2a,
           W_ne_lvl2b, eps_ne_1, eps_ne_2, W_ec_lift1, W_ec_lift2, W_ec_lvl1,
           W_ec_lvl2a, W_ec_lvl2b, eps_ec_11, eps_ec_12, eps_ec_2, W_mlp):
    N = node_rep.shape[0]
    E = edge_rep.shape[0]
    T = cycle_rep.shape[0]
    C = 20000
    BR = 1000
    T2 = 32 * CH * 47
    E2 = 40 * 8192

    ei = edge_index.astype(jnp.int32)
    CPW = E // (32 * CH)
    i0_3d = ei[0].reshape(32, CPW, CH)
    i1_3d = ei[1].reshape(32, CPW, CH)
    i0_3dt = ei[0].reshape(32 * CPW, 1, CH)
    i1_3dt = ei[1].reshape(32 * CPW, 1, CH)
    cee = cycle_entry_edge.astype(jnp.int32)
    dom = cycle_domain.astype(jnp.int32)
    cee_p = jnp.concatenate([cee, jnp.full((T2 - T,), E, jnp.int32)])
    dom_p = jnp.concatenate([dom, jnp.full((T2 - T,), C, jnp.int32)])
    cee3d = cee_p.reshape(T2 // CH, 1, CH)
    dom3d = dom_p.reshape(T2 // CH, 1, CH)
    crep_p = jnp.concatenate(
        [cycle_rep, jnp.zeros((T2 - T, H), jnp.float32)], axis=0)

    epsmat = jnp.broadcast_to(
        jnp.stack([1.0 + eps_ne_1, 1.0 + eps_ne_2, 1.0 + eps_ec_11,
                   1.0 + eps_ec_12, 1.0 + eps_ec_2,
                   jnp.float32(0), jnp.float32(0), jnp.float32(0)])[:, None],
        (8, H))
    eps12_16 = jnp.broadcast_to((1.0 + eps_ec_12)[None], (16,))

    g0, g1 = _sc_gather2(node_rep, i0_3d, i1_3d, E)

    def fn_s1(xs, fs, ws, eps_ref):
        g0b, g1b, eb = xs
        la = g0b + g1b
        y1e = _mm(jnp.concatenate([la, eb], axis=1), ws[0])
        x2 = eb * eps_ref[1:2, :] + la
        y1o = _mm(x2, ws[1])
        return [y1e, y1o]

    y1e, p1e, y1o, p1o = _tc_fused(
        E, BR, [g0, g1, edge_rep], [], [W_ne_lvl1, W_ne_lift1], epsmat,
        fn_s1, [(H, True), (2 * H, True)], "s1_edge_lin")

    def fn_s2(xs, fs, ws, eps_ref):
        h1 = _nrm(xs[0], fs[0])
        y2o = _mm(_nrm(xs[1], fs[1]), ws[0])
        return [h1, y2o]

    h1, y2o, p2o = _tc_fused(
        E, BR, [y1e, y1o], [p1e, p1o], [W_ne_lift2], epsmat,
        fn_s2, [(H, False), (H, True)], "s2_edge_lin")

    nacc = _sc_scatter_nodes(h1, i0_3dt, i1_3dt, E, N)
    node_out = _tc_node(node_rep, nacc, epsmat, W_ne_lvl2a, W_ne_lvl2b, N)

    g_pad, mg, mc, rcp = _sc_seg_means(dom3d, C, T2, gather_src=edge_rep,
                                       cee3d=cee3d, linear_src=crep_p)
    gm_pad, cycb_pad = _sc_bcast(dom3d, [mg, mc], C, T2)

    def fn_s5(xs, fs, ws, eps_ref):
        gb, gmb, cb, cbb = xs
        y1c = _mm(jnp.concatenate([gb, gmb, cb], axis=1), ws[0])
        ev = eps_ref[4:5, :]
        x2 = jnp.concatenate([cb * ev + gb, cbb * ev + gmb], axis=1)
        y1k = _mm(x2, ws[1])
        return [y1c, y1k]

    y1c, p1c, y1k, p1k = _tc_fused(
        T, BR, [g_pad, gm_pad, cycle_rep, cycb_pad], [],
        [W_ec_lvl1, W_ec_lift1], epsmat,
        fn_s5, [(H, True), (2 * H, True)], "s5_cyc_lin")

    def fn_s6(xs, fs, ws, eps_ref):
        h2 = _nrm(xs[0], fs[0])
        y2k = _mm(_nrm(xs[1], fs[1]), ws[0])
        return [h2, y2k]

    h2, y2k, p2k = _tc_fused(
        T, BR, [y1c, y1k], [p1c, p1k], [W_ec_lift2], epsmat,
        fn_s6, [(H, False), (H, True)], "s6_cyc_lin")

    def fn_norm_only(xs, fs, ws, eps_ref):
        return [_nrm(xs[0], fs[0])]

    cycle_out, = _tc_fused(T, BR, [y2k], [p2k], [], epsmat,
                           fn_norm_only, [(H, False)], "s9_cyc_out")

    h2p = jnp.concatenate([h2, jnp.zeros((T2 - T, H), jnp.float32)], axis=0)
    mh2 = _sc_seg_means(dom3d, C, T2, linear_src=h2p, rcp_in=rcp)
    val = _sc_val(dom3d, h2p, mh2, eps12_16, C, T2)
    lvlc = _sc_scatter_edges(val, cee_p, E2, T2)[:E]

    def fn_s10(xs, fs, ws, eps_ref):
        x = xs[0] * eps_ref[2:3, :] + xs[1]
        return [_mm(x, ws[0])]

    y1f, p1f = _tc_fused(E, BR, [edge_rep, lvlc], [], [W_ec_lvl2a], epsmat,
                         fn_s10, [(2 * H, True)], "s10_edge2")

    def fn_s11(xs, fs, ws, eps_ref):
        return [_mm(_nrm(xs[0], fs[0]), ws[0])]

    y2f, p2f = _tc_fused(E, BR, [y1f], [p1f], [W_ec_lvl2b], epsmat,
                         fn_s11, [(H, True)], "s11_edge2")

    def fn_s12(xs, fs, ws, eps_ref):
        x = jnp.concatenate([_nrm(xs[0], fs[0]), _nrm(xs[1], fs[1])], axis=1)
        return [_mm(x, ws[0])]

    z, pz = _tc_fused(E, BR, [y2o, y2f], [p2o, p2f], [W_mlp], epsmat,
                      fn_s12, [(H, True)], "s12_final_lin")

    edge_out, = _tc_fused(E, BR, [z], [pz], [], epsmat,
                          fn_norm_only, [(H, False)], "s12b_final_norm")

    return (node_out, edge_out, cycle_out)


# speedup vs baseline: 1.3472x; 1.0050x over previous
import functools

import jax
import jax.numpy as jnp
from jax import lax
from jax.experimental import pallas as pl
from jax.experimental.pallas import tpu as pltpu, tpu_sc as plsc

EPS = 1e-05
H = 128
CH = 80

_SC_PARAMS = pltpu.CompilerParams(needs_layout_passes=False)


def _mesh():
    return plsc.VectorSubcoreMesh(core_axis_name="c", subcore_axis_name="s")


def _f32(shape):
    return jax.ShapeDtypeStruct(shape, jnp.float32)


def _sc_gather2(table, i0_3d, i1_3d, E):
    CPW = E // (32 * CH)

    @functools.partial(
        pl.kernel,
        out_type=(_f32((E, H)), _f32((E, H))),
        mesh=_mesh(),
        compiler_params=_SC_PARAMS,
        scratch_types=[
            pltpu.VMEM((CPW, CH), jnp.int32),
            pltpu.VMEM((CPW, CH), jnp.int32),
            pltpu.VMEM((2, CH, H), jnp.float32),
            pltpu.VMEM((2, CH, H), jnp.float32),
            pltpu.SemaphoreType.DMA,
            pltpu.SemaphoreType.DMA,
        ],
    )
    def k(tab_h, i0_h, i1_h, g0_h, g1_h, i0v, i1v, b0, b1, s0, s1):
        cid = lax.axis_index("c")
        sid = lax.axis_index("s")
        w = sid * 2 + cid
        r0 = w * CPW
        pltpu.sync_copy(i0_h.at[w], i0v)
        pltpu.sync_copy(i1_h.at[w], i1v)
        pltpu.async_copy(tab_h.at[i0v.at[0]], b0.at[0], s0)
        pltpu.async_copy(tab_h.at[i1v.at[0]], b1.at[0], s1)

        def body(ch, _):
            cur = ch % 2
            pltpu.make_async_copy(g0_h.at[pl.ds(0, CH)], b0.at[cur],
                                  s0).wait()
            pltpu.make_async_copy(g0_h.at[pl.ds(0, CH)], b1.at[cur],
                                  s1).wait()

            @pl.when(ch + 1 < CPW)
            def _():
                nxt = (ch + 1) % 2
                pltpu.async_copy(tab_h.at[i0v.at[ch + 1]], b0.at[nxt], s0)
                pltpu.async_copy(tab_h.at[i1v.at[ch + 1]], b1.at[nxt], s1)

            base = (r0 + ch) * CH
            pltpu.sync_copy(b0.at[cur], g0_h.at[pl.ds(base, CH)])
            pltpu.sync_copy(b1.at[cur], g1_h.at[pl.ds(base, CH)])
            return 0

        lax.fori_loop(0, CPW, body, 0)

    return k(table, i0_3d, i1_3d)


def _sc_scatter_nodes(h1, i0_3d, i1_3d, E, N):
    CPT = E // (16 * CH)
    NH = N // 2
    ACC = 5120
    STR = 312

    @functools.partial(
        pl.kernel,
        out_type=_f32((N, H)),
        mesh=_mesh(),
        compiler_params=_SC_PARAMS,
        scratch_types=[
            pltpu.VMEM((25, 1, CH), jnp.int32),
            pltpu.VMEM((25, 1, CH), jnp.int32),
            pltpu.VMEM((2, CH, H), jnp.float32),
            pltpu.VMEM((2, CH), jnp.int32),
            pltpu.VMEM((64, H), jnp.float32),
            pltpu.VMEM_SHARED((ACC + 8, H), jnp.float32),
            pltpu.SemaphoreType.DMA,
        ],
    )
    def k(h1_h, i0_h, i1_h, out_h, i0v, i1v, hbuf, idxb, zbuf, nacc, sem):
        cid = lax.axis_index("c")
        sid = lax.axis_index("s")
        zv = jnp.zeros((16,), jnp.float32)

        def zrow(r, _):
            for j in range(H // 16):
                zbuf[r, pl.ds(16 * j, 16)] = zv
            return 0

        lax.fori_loop(0, 64, zrow, 0)
        for b in range(5):
            pltpu.sync_copy(zbuf, nacc.at[pl.ds(sid * 320 + b * 64, 64)])

        @pl.when(sid == 0)
        def _():
            pltpu.sync_copy(zbuf.at[pl.ds(0, 8)], nacc.at[pl.ds(ACC, 8)])

        plsc.subcore_barrier()
        nbase = cid * NH

        cp = pltpu.async_copy(h1_h.at[pl.ds(sid * CPT * CH, CH)],
                              hbuf.at[0], sem)

        def body(ch, _):
            g = ch % 25

            @pl.when(g == 0)
            def _():
                pltpu.sync_copy(i0_h.at[pl.ds(sid * CPT + ch, 25)], i0v)
                pltpu.sync_copy(i1_h.at[pl.ds(sid * CPT + ch, 25)], i1v)

            nxt = (ch + 1) % 2
            cur = ch % 2
            pltpu.make_async_copy(h1_h.at[pl.ds(0, CH)], hbuf.at[cur],
                                  sem).wait()

            @pl.when(ch + 1 < CPT)
            def _():
                base2 = (sid * CPT + ch + 1) * CH
                pltpu.async_copy(h1_h.at[pl.ds(base2, CH)], hbuf.at[nxt],
                                 sem)

            for i5 in range(CH // 16):
                sl = pl.ds(16 * i5, 16)
                v0 = i0v[g, 0, sl] - nbase
                v1 = i1v[g, 0, sl] - nbase
                idxb[0, sl] = jnp.where((v0 >= 0) & (v0 < NH), v0, ACC)
                idxb[1, sl] = jnp.where((v1 >= 0) & (v1 < NH), v1, ACC)
            pltpu.sync_copy(hbuf.at[cur], nacc.at[idxb.at[0]], add=True)
            pltpu.sync_copy(hbuf.at[cur], nacc.at[idxb.at[1]], add=True)
            return 0

        lax.fori_loop(0, CPT, body, 0)
        _ = cp
        plsc.subcore_barrier()
        pltpu.sync_copy(nacc.at[pl.ds(sid * STR, STR)],
                        out_h.at[pl.ds(nbase + sid * STR, STR)])

        @pl.when(sid == 15)
        def _():
            pltpu.sync_copy(nacc.at[pl.ds(4992, 8)],
                            out_h.at[pl.ds(nbase + 4992, 8)])

    return k(h1, i0_3d, i1_3d)


def _sc_seg_means(dom3d, C, T2, gather_src=None, cee3d=None, linear_src=None,
                  rcp_in=None):
    NCHK = T2 // CH
    NPT = NCHK // 16
    SEG = C // 2
    SEGH = SEG // 2
    SEGP = 10240
    ACC = 5120
    STR = ACC // 16
    ZB = STR // 4
    do_g = gather_src is not None
    do_l = linear_src is not None
    do_cnt = rcp_in is None

    outs = []
    if do_g:
        outs.append(_f32((T2, H)))
        outs.append(_f32((2 * SEGP, H)))
    if do_l:
        outs.append(_f32((2 * SEGP, H)))
    if do_cnt:
        outs.append(_f32((2 * SEGP, 16)))

    ins = [dom3d]
    if do_g:
        ins += [gather_src, cee3d]
    if do_l:
        ins += [linear_src]
    if not do_cnt:
        ins += [rcp_in]

    @functools.partial(
        pl.kernel,
        out_type=tuple(outs) if len(outs) > 1 else outs[0],
        mesh=_mesh(),
        compiler_params=_SC_PARAMS,
        scratch_types=[
            pltpu.VMEM((NPT, 1, CH), jnp.int32),
            pltpu.VMEM((NPT, 1, CH), jnp.int32),
            pltpu.VMEM((2, CH), jnp.int32),
            pltpu.VMEM((2, CH, H), jnp.float32),
            pltpu.VMEM((CH, H), jnp.float32),
            pltpu.VMEM((ZB, H), jnp.float32),
            pltpu.VMEM((ZB, 16), jnp.float32),
            pltpu.VMEM_SHARED((ACC + 8, H), jnp.float32),
            pltpu.SemaphoreType.DMA,
        ],
    )
    def k(*refs):
        pos = 0
        dom_h = refs[pos]; pos += 1
        if do_g:
            gsrc_h = refs[pos]; pos += 1
            cee_h = refs[pos]; pos += 1
        if do_l:
            lsrc_h = refs[pos]; pos += 1
        if not do_cnt:
            rcp_h = refs[pos]; pos += 1
        if do_g:
            g_h = refs[pos]; pos += 1
            mg_h = refs[pos]; pos += 1
        if do_l:
            ml_h = refs[pos]; pos += 1
        if do_cnt:
            rcp_h = refs[pos]; pos += 1
        lsrc0_h = gsrc_h if do_g else lsrc_h
        (domsl, ceesl, idxb, vbuf, ones128, zbuf, rcpb, sums,
         sem) = refs[pos:pos + 9]

        cid = lax.axis_index("c")
        sid = lax.axis_index("s")
        zv = jnp.zeros((16,), jnp.float32)
        ov = jnp.ones((16,), jnp.float32)

        def initrow(r, _):
            for j in range(H // 16):
                zbuf[r, pl.ds(16 * j, 16)] = zv
            return 0

        lax.fori_loop(0, ZB, initrow, 0)

        def onesrow(r, _):
            for j in range(H // 16):
                ones128[r, pl.ds(16 * j, 16)] = ov
            return 0

        lax.fori_loop(0, CH, onesrow, 0)
        pltpu.sync_copy(dom_h.at[pl.ds(sid * NPT, NPT)], domsl)
        if do_g:
            pltpu.sync_copy(cee_h.at[pl.ds(sid * NPT, NPT)], ceesl)

        def zero_acc():
            for b in range(4):
                pltpu.sync_copy(zbuf, sums.at[pl.ds(sid * STR + b * ZB, ZB)])

            @pl.when(sid == 0)
            def _():
                pltpu.sync_copy(zbuf.at[pl.ds(0, 8)], sums.at[pl.ds(ACC, 8)])

        def build_idx(kk, half):
            dbase = cid * SEG + half * SEGH
            for i5 in range(CH // 16):
                d16 = domsl[kk, 0, pl.ds(16 * i5, 16)]
                dl = d16 - dbase
                ok = (dl >= 0) & (dl < SEGH)
                idxb[0, pl.ds(16 * i5, 16)] = jnp.where(ok, dl, ACC)

        def fetch(kk, buf):
            if phase_is_gather[0]:
                pltpu.async_copy(gsrc_h.at[ceesl.at[kk, 0]], buf, sem)
            else:
                j = sid * NPT + kk
                pltpu.async_copy(lsrc_h.at[pl.ds(j * CH, CH)], buf, sem)

        phase_is_gather = [False]

        def accumulate(phase, half):
            if phase == 2:
                def body2(kk, _):
                    build_idx(kk, half)
                    pltpu.sync_copy(ones128, sums.at[idxb.at[0]], add=True)
                    return 0

                lax.fori_loop(0, NPT, body2, 0)
                return

            phase_is_gather[0] = phase == 0
            fetch(0, vbuf.at[0])

            def body(kk, _):
                cur = kk % 2
                pltpu.make_async_copy(lsrc0_h.at[pl.ds(0, CH)],
                                      vbuf.at[cur], sem).wait()

                @pl.when(kk + 1 < NPT)
                def _():
                    fetch(kk + 1, vbuf.at[(kk + 1) % 2])

                if phase == 0:
                    j = sid * NPT + kk

                    @pl.when((j % 2) == cid)
                    def _():
                        pltpu.sync_copy(vbuf.at[cur],
                                        g_h.at[pl.ds(j * CH, CH)])
                build_idx(kk, half)
                pltpu.sync_copy(vbuf.at[cur], sums.at[idxb.at[0]], add=True)
                return 0

            lax.fori_loop(0, NPT, body, 0)

        def out_row0(half, b):
            return cid * SEGP + half * SEGH + sid * STR + b * ZB

        def finalize_counts(half):
            for b in range(4):
                r0 = sid * STR + b * ZB
                pltpu.sync_copy(sums.at[pl.ds(r0, ZB)], zbuf)

                def frow(r, _):
                    c16 = zbuf[r, pl.ds(0, 16)]
                    rcpb[r, pl.ds(0, 16)] = 1.0 / jnp.maximum(c16, 1.0)
                    return 0

                lax.fori_loop(0, ZB, frow, 0)
                pltpu.sync_copy(rcpb, rcp_h.at[pl.ds(out_row0(half, b), ZB)])
            lax.fori_loop(0, ZB, initrow, 0)

        def finalize(m_h, half):
            for b in range(4):
                r0 = sid * STR + b * ZB
                pltpu.sync_copy(sums.at[pl.ds(r0, ZB)], zbuf)
                pltpu.sync_copy(rcp_h.at[pl.ds(out_row0(half, b), ZB)], rcpb)

                def frow(r, _):
                    rcp = rcpb[r, pl.ds(0, 16)]
                    for j in range(H // 16):
                        zbuf[r, pl.ds(16 * j, 16)] = \
                            zbuf[r, pl.ds(16 * j, 16)] * rcp
                    return 0

                lax.fori_loop(0, ZB, frow, 0)
                pltpu.sync_copy(zbuf, m_h.at[pl.ds(out_row0(half, b), ZB)])
            lax.fori_loop(0, ZB, initrow, 0)

        plan = []
        if do_cnt:
            plan += [(2, None)]
        if do_g:
            plan += [(0, mg_h)]
        if do_l:
            plan += [(1, ml_h)]
        first = True
        for phase, m_h in plan:
            for half in (0, 1):
                if not first:
                    plsc.subcore_barrier()
                zero_acc()
                plsc.subcore_barrier()
                accumulate(phase, half)
                plsc.subcore_barrier()
                if phase == 2:
                    finalize_counts(half)
                else:
                    finalize(m_h, half)
                first = False

    return k(*ins)


def _sc_bcast(dom3d, means_list, C, T2):
    NCHK = T2 // CH
    NPW = NCHK // 32
    SEG, PAD = 10000, 240
    nm = len(means_list)

    @functools.partial(
        pl.kernel,
        out_type=tuple(_f32((T2, H)) for _ in range(nm)) if nm > 1
        else _f32((T2, H)),
        mesh=_mesh(),
        compiler_params=_SC_PARAMS,
        scratch_types=[
            pltpu.VMEM((NPW, 1, CH), jnp.int32),
            pltpu.VMEM((2, CH), jnp.int32),
            pltpu.VMEM((2, 2, CH, H), jnp.float32),
            pltpu.SemaphoreType.DMA,
        ],
    )
    def k(*refs):
        dom_h = refs[0]
        m_hs = refs[1:1 + nm]
        o_hs = refs[1 + nm:1 + 2 * nm]
        domsl, idxb, vbuf, sem = refs[1 + 2 * nm:]
        cid = lax.axis_index("c")
        sid = lax.axis_index("s")
        w = sid * 2 + cid
        pltpu.sync_copy(dom_h.at[pl.ds(w * NPW, NPW)], domsl)

        def build(kk, slot):
            for i5 in range(CH // 16):
                d16 = domsl[kk, 0, pl.ds(16 * i5, 16)]
                idxb[slot, pl.ds(16 * i5, 16)] = \
                    d16 + jnp.where(d16 >= SEG, PAD, 0)

        def fire(kk, slot):
            for mi in range(nm):
                pltpu.async_copy(m_hs[mi].at[idxb.at[slot]],
                                 vbuf.at[slot, mi], sem)

        build(0, 0)
        fire(0, 0)

        def body(kk, _):
            cur = kk % 2
            for mi in range(nm):
                pltpu.make_async_copy(o_hs[0].at[pl.ds(0, CH)],
                                      vbuf.at[cur, mi], sem).wait()

            @pl.when(kk + 1 < NPW)
            def _():
                build(kk + 1, (kk + 1) % 2)
                fire(kk + 1, (kk + 1) % 2)

            j = w * NPW + kk
            for mi in range(nm):
                pltpu.sync_copy(vbuf.at[cur, mi],
                                o_hs[mi].at[pl.ds(j * CH, CH)])
            return 0

        lax.fori_loop(0, NPW, body, 0)

    return k(dom3d, *means_list)


def _sc_val(dom3d, h2p, mh2, scale16, C, T2):
    NCHK = T2 // CH
    NPW = NCHK // 32
    SEG, PAD = 10000, 240

    @functools.partial(
        pl.kernel,
        out_type=_f32((T2, H)),
        mesh=_mesh(),
        compiler_params=_SC_PARAMS,
        scratch_types=[
            pltpu.VMEM((NPW, 1, CH), jnp.int32),
            pltpu.VMEM((2, CH), jnp.int32),
            pltpu.VMEM((2, CH, H), jnp.float32),
            pltpu.VMEM((2, CH, H), jnp.float32),
            pltpu.VMEM((16,), jnp.float32),
            pltpu.SemaphoreType.DMA,
            pltpu.SemaphoreType.DMA,
        ],
    )
    def k(dom_h, h2_h, m_h, sc_h, out_h, domsl, idxb, b1, b2, scv, s1, s2):
        cid = lax.axis_index("c")
        sid = lax.axis_index("s")
        w = sid * 2 + cid
        pltpu.sync_copy(sc_h, scv)
        ev = scv[pl.ds(0, 16)]
        pltpu.sync_copy(dom_h.at[pl.ds(w * NPW, NPW)], domsl)

        def build(kk, slot):
            for i5 in range(CH // 16):
                d16 = domsl[kk, 0, pl.ds(16 * i5, 16)]
                idxb[slot, pl.ds(16 * i5, 16)] = \
                    d16 + jnp.where(d16 >= SEG, PAD, 0)

        def fire(kk, slot):
            j = w * NPW + kk
            pltpu.async_copy(h2_h.at[pl.ds(j * CH, CH)], b1.at[slot], s1)
            pltpu.async_copy(m_h.at[idxb.at[slot]], b2.at[slot], s2)

        build(0, 0)
        fire(0, 0)

        def body(kk, _):
            cur = kk % 2
            pltpu.make_async_copy(h2_h.at[pl.ds(0, CH)], b1.at[cur],
                                  s1).wait()
            pltpu.make_async_copy(h2_h.at[pl.ds(0, CH)], b2.at[cur],
                                  s2).wait()

            @pl.when(kk + 1 < NPW)
            def _():
                build(kk + 1, (kk + 1) % 2)
                fire(kk + 1, (kk + 1) % 2)

            def crow(r, _):
                for j8 in range(H // 16):
                    sl = pl.ds(16 * j8, 16)
                    b1[cur, r, sl] = b1[cur, r, sl] * ev + b2[cur, r, sl]
                return 0

            lax.fori_loop(0, CH, crow, 0)
            j = w * NPW + kk
            pltpu.sync_copy(b1.at[cur], out_h.at[pl.ds(j * CH, CH)])
            return 0

        lax.fori_loop(0, NPW, body, 0)

    return k(dom3d, h2p, mh2, scale16)


def _sc_scatter_edges(val, cee_flat, E2, T2):
    RNG = 8192
    NPSC = E2 // RNG // 2
    TPT = T2 // 16
    NIT = TPT // 16
    ACC = 8320
    DUMMY = RNG + 8
    GCH = 128
    LSZ = TPT + 2 * GCH

    @functools.partial(
        pl.kernel,
        out_type=_f32((E2, H)),
        mesh=_mesh(),
        compiler_params=_SC_PARAMS,
        scratch_types=[
            pltpu.VMEM((TPT,), jnp.int32),
            pltpu.VMEM((LSZ,), jnp.int32),
            pltpu.VMEM((LSZ // GCH + 1, GCH), jnp.int32),
            pltpu.VMEM((GCH, H), jnp.float32),
            pltpu.VMEM((GCH, H), jnp.float32),
            pltpu.VMEM_SHARED((ACC, H), jnp.float32),
            pltpu.SemaphoreType.DMA,
        ],
    )
    def k(val_h, cee_h, out_h, ceebuf, tlist, e2d, vbuf, zbuf, acc, sem):
        cid = lax.axis_index("c")
        sid = lax.axis_index("s")
        zv = jnp.zeros((16,), jnp.float32)
        iota = lax.iota(jnp.int32, 16)

        def zrow(r, _):
            for j in range(H // 16):
                zbuf[r, pl.ds(16 * j, 16)] = zv
            return 0

        lax.fori_loop(0, GCH, zrow, 0)
        tb = sid * TPT
        pltpu.sync_copy(cee_h.at[pl.ds(tb, TPT)], ceebuf)
        for b in range(4):
            pltpu.sync_copy(zbuf, acc.at[pl.ds(sid * 520 + b * GCH, GCH)])
        pltpu.sync_copy(zbuf.at[pl.ds(0, 8)],
                        acc.at[pl.ds(sid * 520 + 512, 8)])
        plsc.subcore_barrier()

        def one_pass(p, _):
            base = (cid * NPSC + p) * RNG

            def scan(i, m):
                ev16 = ceebuf[pl.ds(16 * i, 16)]
                el = ev16 - base
                ok = (el >= 0) & (el < RNG)
                c16 = plsc.all_reduce_population_count(ok)
                plsc.store_compressed(tlist.at[pl.ds(m, 16)],
                                      16 * i + iota, mask=ok)
                return m + c16[0]

            m = lax.fori_loop(0, NIT, scan, jnp.int32(0))
            for g5 in range(GCH // 16):
                tlist[pl.ds(m + 16 * g5, 16)] = jnp.zeros((16,), jnp.int32)
            nch = (m + GCH - 1) // GCH

            def copy2d(ch2, _):
                for i5 in range(GCH // 16):
                    pos = GCH * ch2 + 16 * i5
                    tloc = tlist[pl.ds(pos, 16)]
                    ee = plsc.load_gather(ceebuf, [tloc]) - base
                    valid = (pos + iota) < m
                    e2d[ch2, pl.ds(16 * i5, 16)] = \
                        jnp.where(valid, ee, DUMMY)
                    tlist[pl.ds(pos, 16)] = tloc + tb
                return 0

            lax.fori_loop(0, nch, copy2d, 0)

            def gsc(ch2, _):
                pltpu.async_copy(
                    val_h.at[tlist.at[pl.ds(GCH * ch2, GCH)]],
                    vbuf, sem).wait()
                pltpu.sync_copy(vbuf, acc.at[e2d.at[ch2]], add=True)
                return 0

            lax.fori_loop(0, nch, gsc, 0)
            plsc.subcore_barrier()
            wr = RNG // 16
            pltpu.sync_copy(acc.at[pl.ds(sid * wr, wr)],
                            out_h.at[pl.ds(base + sid * wr, wr)])
            plsc.subcore_barrier()

            def tz(ch2, _):
                pltpu.sync_copy(zbuf, acc.at[e2d.at[ch2]])
                return 0

            lax.fori_loop(0, nch, tz, 0)
            plsc.subcore_barrier()
            return 0

        lax.fori_loop(0, NPSC, one_pass, 0)

    return k(val, cee_flat)


def _tc_fused(R, BR, ins, stats, weights, epsmat, fn, outs_spec, name):
    nb = R // BR
    n_in, n_st, n_w = len(ins), len(stats), len(weights)

    def body(*refs):
        i = pl.program_id(0)
        in_refs = refs[:n_in]
        st_refs = refs[n_in:n_in + n_st]
        w_refs = refs[n_in + n_st:n_in + n_st + n_w]
        eps_ref = refs[n_in + n_st + n_w]
        rest = refs[n_in + n_st + n_w + 1:]
        n_o = len(outs_spec) + sum(1 for _, ws in outs_spec if ws)
        out_refs = rest[:n_o]
        scr_refs = rest[n_o:]

        @pl.when(i == 0)
        def _():
            for st_ref, scr in zip(st_refs, scr_refs):
                s = jnp.sum(st_ref[...], axis=0)
                mu = s[0:1] / R
                var = s[1:2] / R - mu * mu
                rs = lax.rsqrt(var + EPS)
                scr[0:1, :] = mu
                scr[1:2, :] = rs

        finstats = [(scr[0:1, :], scr[1:2, :]) for scr in scr_refs]
        outs = fn([r[...] for r in in_refs], finstats,
                  [r[...] for r in w_refs], eps_ref)
        oi = 0
        for o, (ko, ws) in zip(outs, outs_spec):
            out_refs[oi][...] = o
            oi += 1
            if ws:
                out_refs[oi][0, 0, :] = jnp.sum(o, axis=0)
                out_refs[oi][0, 1, :] = jnp.sum(o * o, axis=0)
                oi += 1

    in_specs = (
        [pl.BlockSpec((BR, a.shape[1]), lambda i: (i, 0)) for a in ins]
        + [pl.BlockSpec(p.shape, lambda i: (0, 0, 0)) for p in stats]
        + [pl.BlockSpec(w.shape, lambda i: (0, 0)) for w in weights]
        + [pl.BlockSpec(epsmat.shape, lambda i: (0, 0))]
    )
    out_shape, out_specs = [], []
    for ko, ws in outs_spec:
        out_shape.append(_f32((R, ko)))
        out_specs.append(pl.BlockSpec((BR, ko), lambda i: (i, 0)))
        if ws:
            out_shape.append(_f32((nb, 2, ko)))
            out_specs.append(pl.BlockSpec((1, 2, ko), lambda i: (i, 0, 0)))
    scratch = [pltpu.VMEM((2, p.shape[2]), jnp.float32) for p in stats]
    return pl.pallas_call(
        body,
        grid=(nb,),
        in_specs=in_specs,
        out_specs=out_specs,
        out_shape=out_shape,
        scratch_shapes=scratch,
        name=name,
    )(*ins, *stats, *weights, epsmat)


def _nrm(y, st):
    mu, rs = st
    return jnp.maximum((y - mu) * rs, 0.0)


def _mm(x, w):
    return lax.dot_general(x, w, (((1,), (1,)), ((), ())),
                           preferred_element_type=jnp.float32)


def _tc_node(node_rep, partials, epsmat, Wa, Wb, N):
    def body(x_ref, p_ref, eps_ref, wa_ref, wb_ref, o_ref):
        ev = eps_ref[0:1, :]
        x = x_ref[...] * ev + p_ref[...]
        y1 = _mm(x, wa_ref[...])
        mu = jnp.mean(y1, axis=0, keepdims=True)
        var = jnp.mean((y1 - mu) ** 2, axis=0, keepdims=True)
        h = jnp.maximum((y1 - mu) * lax.rsqrt(var + EPS), 0.0)
        y2 = _mm(h, wb_ref[...])
        mu2 = jnp.mean(y2, axis=0, keepdims=True)
        var2 = jnp.mean((y2 - mu2) ** 2, axis=0, keepdims=True)
        o_ref[...] = jnp.maximum((y2 - mu2) * lax.rsqrt(var2 + EPS), 0.0)

    return pl.pallas_call(
        body,
        out_shape=_f32((N, H)),
        name="node_mlp2",
    )(node_rep, partials, epsmat, Wa, Wb)


def kernel(node_rep, edge_rep, cycle_rep, edge_index, cycle_entry_edge,
           cycle_domain, W_ne_lift1, W_ne_lift2, W_ne_lvl1, W_ne_lvl2a,
           W_ne_lvl2b, eps_ne_1, eps_ne_2, W_ec_lift1, W_ec_lift2, W_ec_lvl1,
           W_ec_lvl2a, W_ec_lvl2b, eps_ec_11, eps_ec_12, eps_ec_2, W_mlp):
    N = node_rep.shape[0]
    E = edge_rep.shape[0]
    T = cycle_rep.shape[0]
    C = 20000
    BR = 1000
    T2 = 32 * CH * 47
    E2 = 40 * 8192

    ei = edge_index.astype(jnp.int32)
    CPW = E // (32 * CH)
    i0_3d = ei[0].reshape(32, CPW, CH)
    i1_3d = ei[1].reshape(32, CPW, CH)
    i0_3dt = ei[0].reshape(32 * CPW, 1, CH)
    i1_3dt = ei[1].reshape(32 * CPW, 1, CH)
    cee = cycle_entry_edge.astype(jnp.int32)
    dom = cycle_domain.astype(jnp.int32)
    cee_p = jnp.concatenate([cee, jnp.full((T2 - T,), E, jnp.int32)])
    dom_p = jnp.concatenate([dom, jnp.full((T2 - T,), C, jnp.int32)])
    cee3d = cee_p.reshape(T2 // CH, 1, CH)
    dom3d = dom_p.reshape(T2 // CH, 1, CH)
    crep_p = jnp.concatenate(
        [cycle_rep, jnp.zeros((T2 - T, H), jnp.float32)], axis=0)

    epsmat = jnp.broadcast_to(
        jnp.stack([1.0 + eps_ne_1, 1.0 + eps_ne_2, 1.0 + eps_ec_11,
                   1.0 + eps_ec_12, 1.0 + eps_ec_2,
                   jnp.float32(0), jnp.float32(0), jnp.float32(0)])[:, None],
        (8, H))
    eps12_16 = jnp.broadcast_to((1.0 + eps_ec_12)[None], (16,))

    g0, g1 = _sc_gather2(node_rep, i0_3d, i1_3d, E)

    def fn_s1(xs, fs, ws, eps_ref):
        g0b, g1b, eb = xs
        la = g0b + g1b
        y1e = _mm(jnp.concatenate([la, eb], axis=1), ws[0])
        x2 = eb * eps_ref[1:2, :] + la
        y1o = _mm(x2, ws[1])
        return [y1e, y1o]

    y1e, p1e, y1o, p1o = _tc_fused(
        E, BR, [g0, g1, edge_rep], [], [W_ne_lvl1, W_ne_lift1], epsmat,
        fn_s1, [(H, True), (2 * H, True)], "s1_edge_lin")

    def fn_s2(xs, fs, ws, eps_ref):
        h1 = _nrm(xs[0], fs[0])
        y2o = _mm(_nrm(xs[1], fs[1]), ws[0])
        return [h1, y2o]

    h1, y2o, p2o = _tc_fused(
        E, BR, [y1e, y1o], [p1e, p1o], [W_ne_lift2], epsmat,
        fn_s2, [(H, False), (H, True)], "s2_edge_lin")

    nacc = _sc_scatter_nodes(h1, i0_3dt, i1_3dt, E, N)
    node_out = _tc_node(node_rep, nacc, epsmat, W_ne_lvl2a, W_ne_lvl2b, N)

    g_pad, mg, mc, rcp = _sc_seg_means(dom3d, C, T2, gather_src=edge_rep,
                                       cee3d=cee3d, linear_src=crep_p)
    gm_pad, cycb_pad = _sc_bcast(dom3d, [mg, mc], C, T2)

    def fn_s5(xs, fs, ws, eps_ref):
        gb, gmb, cb, cbb = xs
        y1c = _mm(jnp.concatenate([gb, gmb, cb], axis=1), ws[0])
        ev = eps_ref[4:5, :]
        x2 = jnp.concatenate([cb * ev + gb, cbb * ev + gmb], axis=1)
        y1k = _mm(x2, ws[1])
        return [y1c, y1k]

    y1c, p1c, y1k, p1k = _tc_fused(
        T, BR, [g_pad, gm_pad, cycle_rep, cycb_pad], [],
        [W_ec_lvl1, W_ec_lift1], epsmat,
        fn_s5, [(H, True), (2 * H, True)], "s5_cyc_lin")

    def fn_s6(xs, fs, ws, eps_ref):
        h2 = _nrm(xs[0], fs[0])
        y2k = _mm(_nrm(xs[1], fs[1]), ws[0])
        return [h2, y2k]

    h2, y2k, p2k = _tc_fused(
        T, BR, [y1c, y1k], [p1c, p1k], [W_ec_lift2], epsmat,
        fn_s6, [(H, False), (H, True)], "s6_cyc_lin")

    def fn_norm_only(xs, fs, ws, eps_ref):
        return [_nrm(xs[0], fs[0])]

    cycle_out, = _tc_fused(T, BR, [y2k], [p2k], [], epsmat,
                           fn_norm_only, [(H, False)], "s9_cyc_out")

    h2p = jnp.concatenate([h2, jnp.zeros((T2 - T, H), jnp.float32)], axis=0)
    mh2 = _sc_seg_means(dom3d, C, T2, linear_src=h2p, rcp_in=rcp)
    val = _sc_val(dom3d, h2p, mh2, eps12_16, C, T2)
    lvlc = _sc_scatter_edges(val, cee_p, E2, T2)[:E]

    def fn_s10(xs, fs, ws, eps_ref):
        x = xs[0] * eps_ref[2:3, :] + xs[1]
        return [_mm(x, ws[0])]

    y1f, p1f = _tc_fused(E, BR, [edge_rep, lvlc], [], [W_ec_lvl2a], epsmat,
                         fn_s10, [(2 * H, True)], "s10_edge2")

    def fn_s11(xs, fs, ws, eps_ref):
        return [_mm(_nrm(xs[0], fs[0]), ws[0])]

    y2f, p2f = _tc_fused(E, BR, [y1f], [p1f], [W_ec_lvl2b], epsmat,
                         fn_s11, [(H, True)], "s11_edge2")

    def fn_s12(xs, fs, ws, eps_ref):
        x = jnp.concatenate([_nrm(xs[0], fs[0]), _nrm(xs[1], fs[1])], axis=1)
        return [_mm(x, ws[0])]

    z, pz = _tc_fused(E, BR, [y2o, y2f], [p2o, p2f], [W_mlp], epsmat,
                      fn_s12, [(H, True)], "s12_final_lin")

    edge_out, = _tc_fused(E, BR, [z], [pz], [], epsmat,
                          fn_norm_only, [(H, False)], "s12b_final_norm")

    return (node_out, edge_out, cycle_out)
